# baseline (speedup 1.0000x reference)
import functools

import jax
import jax.numpy as jnp
from jax import lax
from jax.experimental import pallas as pl
from jax.experimental.pallas import tpu as pltpu
from jax.experimental.pallas import tpu_sc as plsc

N = 10000
E = 320000
G = 64
NB = 10
BLK = 1000
C = 80
NSUB = 16
F32 = jnp.float32

@functools.lru_cache(maxsize=None)
def _mesh():
    return plsc.VectorSubcoreMesh(core_axis_name="c", subcore_axis_name="s")


def _deg_body(dst_hbm, deg_a, deg_b, dst_v, ones_v, zbuf, acc):
    cid = lax.axis_index("c")
    sid = lax.axis_index("s")
    for j in range(C // 16):
        ones_v[pl.ds(j * 16, 16)] = jnp.ones((16,), F32)
    for j in range(640 // 16):
        zbuf[pl.ds(j * 16, 16)] = jnp.zeros((16,), F32)

    @pl.when(sid < 15)
    def _():
        pltpu.sync_copy(zbuf, acc.at[pl.ds(sid * 640, 640)])

    @pl.when(sid == 15)
    def _():
        pltpu.sync_copy(zbuf.at[pl.ds(0, 400)], acc.at[pl.ds(9600, 400)])

    plsc.subcore_barrier()

    def step(k, carry):
        base = cid * (E // 2) + sid * (E // 2 // NSUB) + k * C
        pltpu.sync_copy(dst_hbm.at[pl.ds(base, C)], dst_v)
        pltpu.sync_copy(ones_v, acc.at[dst_v], add=True)
        return carry

    lax.fori_loop(0, E // 2 // NSUB // C, step, 0)
    plsc.subcore_barrier()

    def dump(out_ref, n):
        pltpu.sync_copy(acc.at[pl.ds(sid * 640, n)], zbuf.at[pl.ds(0, n)])
        pltpu.sync_copy(zbuf.at[pl.ds(0, n)], out_ref.at[pl.ds(sid * 640, n)])

    @pl.when(cid == 0)
    def _():
        @pl.when(sid < 15)
        def _():
            dump(deg_a, 640)

        @pl.when(sid == 15)
        def _():
            dump(deg_a, 400)

    @pl.when(cid == 1)
    def _():
        @pl.when(sid < 15)
        def _():
            dump(deg_b, 640)

        @pl.when(sid == 15)
        def _():
            dump(deg_b, 400)


def _deg_call(dst):
    return pl.kernel(
        _deg_body,
        out_type=[jax.ShapeDtypeStruct((N,), F32),
                  jax.ShapeDtypeStruct((N,), F32)],
        mesh=_mesh(),
        scratch_types=[
            pltpu.VMEM((C,), jnp.int32),
            pltpu.VMEM((C,), F32),
            pltpu.VMEM((640,), F32),
            pltpu.VMEM_SHARED((N,), F32),
        ],
    )(dst)


NRING = 4


def _edge_pipeline(y_hbm, idx_hbm, acc, bufs, yoff, ebase, nchunks, do_off):
    (src_hbm, dst_hbm) = idx_hbm
    (src_v, dst_v, rows_v, isem, gsem, ssem) = bufs
    if do_off:
        off = jnp.zeros((16,), jnp.int32) + yoff

    def start_idx(p, k):
        base = ebase + k * C
        pltpu.async_copy(src_hbm.at[pl.ds(base, C)], src_v[p], isem[p])
        pltpu.async_copy(dst_hbm.at[pl.ds(base, C)], dst_v[p], isem[p])

    def launch_gather(p, k):
        base = ebase + k * C
        pltpu.make_async_copy(src_hbm.at[pl.ds(base, C)], src_v[p],
                              isem[p]).wait()
        pltpu.make_async_copy(dst_hbm.at[pl.ds(base, C)], dst_v[p],
                              isem[p]).wait()
        if do_off:
            for j in range(C // 16):
                src_v[p][pl.ds(j * 16, 16)] = src_v[p][pl.ds(j * 16, 16)] + off
        pltpu.async_copy(y_hbm.at[src_v[p]], rows_v[p], gsem[p])

    def wait_gather(p):
        pltpu.make_async_copy(y_hbm.at[src_v[p]], rows_v[p], gsem[p]).wait()

    def start_scatter(p):
        pltpu.async_copy(rows_v[p], acc.at[dst_v[p]], ssem[p], add=True)

    def wait_scatter(p):
        pltpu.make_async_copy(rows_v[p], acc.at[dst_v[p]], ssem[p]).wait()

    ngroups, rem = divmod(nchunks, NRING)
    assert ngroups >= 1

    def group(t, carry):
        for p in range(NRING):
            k = NRING * t + p

            @pl.when(t >= 1)
            def _():
                wait_scatter(p)

            start_idx(p, k)
            q = (p - 1) % NRING
            if p == 0:
                @pl.when(t >= 1)
                def _():
                    wait_gather(q)
                    start_scatter(q)
            else:
                wait_gather(q)
                start_scatter(q)
            launch_gather(p, k)
        return carry

    lax.fori_loop(0, ngroups, group, 0)
    for r in range(rem):
        k = ngroups * NRING + r
        wait_scatter(r)
        start_idx(r, k)
        q = (r - 1) % NRING
        wait_gather(q)
        start_scatter(q)
        launch_gather(r, k)
    p_last = (nchunks - 1) % NRING
    wait_gather(p_last)
    start_scatter(p_last)
    for p in range(NRING):
        wait_scatter(p)


def _stage_rows(nch, inner):
    def body(t, carry):
        inner(t)
        return carry

    lax.fori_loop(0, nch, body, 0)


def _make_scat(dh):
    def body(y_hbm, src_hbm, dst_hbm, out_hbm, *scr):
        src_v, dst_v, rows_v = scr[0:4], scr[4:8], scr[8:12]
        acc = scr[12]
        isem, gsem, ssem = scr[13:17], scr[17:21], scr[21:25]
        rows_a = rows_v[0]
        cid = lax.axis_index("c")
        sid = lax.axis_index("s")
        yoff = cid * N

        nch = jnp.where(sid == 15, 5, 8)

        def icopy(t):
            r0 = sid * 640 + t * C
            pltpu.sync_copy(y_hbm.at[pl.ds(yoff + r0, C)], rows_a)
            pltpu.sync_copy(rows_a, acc.at[pl.ds(r0, C)])

        _stage_rows(nch, icopy)
        plsc.subcore_barrier()

        bufs = (src_v, dst_v, rows_v, isem, gsem, ssem)
        _edge_pipeline(y_hbm, (src_hbm, dst_hbm), acc, bufs, yoff,
                       sid * (E // NSUB), E // NSUB // C, True)
        plsc.subcore_barrier()

        def ocopy(t):
            r0 = sid * 640 + t * C
            pltpu.sync_copy(acc.at[pl.ds(r0, C)], rows_a)
            pltpu.sync_copy(rows_a, out_hbm.at[pl.ds(yoff + r0, C)])

        _stage_rows(nch, ocopy)

    def run(y, src, dst):
        return pl.kernel(
            body,
            out_type=jax.ShapeDtypeStruct((2 * N, dh), F32),
            mesh=_mesh(),
            scratch_types=(
                [pltpu.VMEM((C,), jnp.int32)] * (2 * NRING)
                + [pltpu.VMEM((C, dh), F32)] * NRING
                + [pltpu.VMEM_SHARED((N, dh), F32)]
                + [pltpu.SemaphoreType.DMA] * (3 * NRING)
            ),
        )(y, src, dst)

    return run


_scat128 = _make_scat(128)


def _scat_edge_body(y_hbm, src_hbm, dst_hbm, out_a, out_b, *scr):
    src_v, dst_v, rows_v = scr[0:4], scr[4:8], scr[8:12]
    acc = scr[12]
    isem, gsem, ssem = scr[13:17], scr[17:21], scr[21:25]
    rows_a = rows_v[0]
    cid = lax.axis_index("c")
    sid = lax.axis_index("s")
    nch = jnp.where(sid == 15, 5, 8)

    @pl.when(cid == 1)
    def _():
        def zrow(r, carry):
            for j in range(128 // 16):
                rows_a[r, pl.ds(j * 16, 16)] = jnp.zeros((16,), F32)
            return carry

        lax.fori_loop(0, C, zrow, 0)

    def icopy(t):
        r0 = sid * 640 + t * C

        @pl.when(cid == 0)
        def _():
            pltpu.sync_copy(y_hbm.at[pl.ds(r0, C)], rows_a)

        pltpu.sync_copy(rows_a, acc.at[pl.ds(r0, C)])

    _stage_rows(nch, icopy)
    plsc.subcore_barrier()

    bufs = (src_v, dst_v, rows_v, isem, gsem, ssem)
    _edge_pipeline(y_hbm, (src_hbm, dst_hbm), acc, bufs, 0,
                   cid * (E // 2) + sid * (E // 2 // NSUB),
                   E // 2 // NSUB // C, False)
    plsc.subcore_barrier()

    def dump(out_ref):
        def ocopy(t):
            r0 = sid * 640 + t * C
            pltpu.sync_copy(acc.at[pl.ds(r0, C)], rows_a)
            pltpu.sync_copy(rows_a, out_ref.at[pl.ds(r0, C)])

        _stage_rows(nch, ocopy)

    @pl.when(cid == 0)
    def _():
        dump(out_a)

    @pl.when(cid == 1)
    def _():
        dump(out_b)


def _scat_edge(y, src, dst):
    return pl.kernel(
        _scat_edge_body,
        out_type=[jax.ShapeDtypeStruct((N, 128), F32),
                  jax.ShapeDtypeStruct((N, 128), F32)],
        mesh=_mesh(),
        scratch_types=(
            [pltpu.VMEM((C,), jnp.int32)] * (2 * NRING)
            + [pltpu.VMEM((C, 128), F32)] * NRING
            + [pltpu.VMEM_SHARED((N, 128), F32)]
            + [pltpu.SemaphoreType.DMA] * (3 * NRING)
        ),
    )(y, src, dst)


def _a1_body(x_ref, w_ref, da_ref, db_ref, y_ref, dinv_ref):
    dinv = lax.rsqrt(da_ref[...] + db_ref[...] + 1.0)
    dinv_ref[...] = dinv
    y_ref[...] = jnp.dot(x_ref[...] * dinv, w_ref[...],
                         preferred_element_type=F32, precision=lax.Precision.HIGHEST)


def _run_a1(x, W1, deg_a, deg_b):
    return pl.pallas_call(
        _a1_body,
        grid=(2, NB),
        in_specs=[
            pl.BlockSpec((BLK, 128), lambda h, i: (i, 0)),
            pl.BlockSpec((128, 128), lambda h, i: (0, h)),
            pl.BlockSpec((BLK, 1), lambda h, i: (i, 0)),
            pl.BlockSpec((BLK, 1), lambda h, i: (i, 0)),
        ],
        out_specs=[
            pl.BlockSpec((BLK, 128), lambda h, i: (h * NB + i, 0)),
            pl.BlockSpec((BLK, 1), lambda h, i: (i, 0)),
        ],
        out_shape=[
            jax.ShapeDtypeStruct((2 * N, 128), F32),
            jax.ShapeDtypeStruct((N, 1), F32),
        ],
    )(x, W1, deg_a, deg_b)


def _onehot(bcol, n_rows):
    iota = lax.broadcasted_iota(jnp.int32, (n_rows, G), 1).astype(F32)
    return (bcol == iota).astype(F32)


def _make_comb_body(hdim, with_cnt, mode):
    def body(s0_ref, s1_ref, dinv_ref, b_ref, bf_ref, h_ref, S1_ref, S2_ref,
             *maybe_S0):
        i = pl.program_id(0)
        if mode == "cat":
            s = jnp.concatenate([s0_ref[...], s1_ref[...]], axis=1)
        else:
            s = s0_ref[...] + s1_ref[...]
        h = s * dinv_ref[...] + b_ref[...]
        h_ref[...] = h
        A = _onehot(bf_ref[...], BLK)
        dn = (((0,), (0,)), ((), ()))
        p1 = lax.dot_general(A, h, dn, preferred_element_type=F32, precision=lax.Precision.HIGHEST)
        p2 = lax.dot_general(A, h * h, dn, preferred_element_type=F32, precision=lax.Precision.HIGHEST)

        @pl.when(i == 0)
        def _():
            S1_ref[...] = jnp.zeros((G, hdim), F32)
            S2_ref[...] = jnp.zeros((G, hdim), F32)
            if with_cnt:
                maybe_S0[0][...] = jnp.zeros((G, 128), F32)

        S1_ref[...] += p1
        S2_ref[...] += p2
        if with_cnt:
            p0 = lax.dot_general(A, jnp.ones((BLK, 128), F32), dn,
                                 preferred_element_type=F32, precision=lax.Precision.HIGHEST)
            maybe_S0[0][...] += p0

    return body


def _run_comb(sa, sb, dinv, bvec, batch_f, hdim, with_cnt, mode):
    if mode == "cat":
        w = hdim // 2
        map_a = lambda i: (i, 0)
        map_b = lambda i: (NB + i, 0)
    else:
        w = hdim
        map_a = lambda i: (i, 0)
        map_b = lambda i: (i, 0)
    out_shape = [
        jax.ShapeDtypeStruct((N, hdim), F32),
        jax.ShapeDtypeStruct((G, hdim), F32),
        jax.ShapeDtypeStruct((G, hdim), F32),
    ]
    out_specs = [
        pl.BlockSpec((BLK, hdim), lambda i: (i, 0)),
        pl.BlockSpec((G, hdim), lambda i: (0, 0)),
        pl.BlockSpec((G, hdim), lambda i: (0, 0)),
    ]
    if with_cnt:
        out_shape.append(jax.ShapeDtypeStruct((G, 128), F32))
        out_specs.append(pl.BlockSpec((G, 128), lambda i: (0, 0)))
    return pl.pallas_call(
        _make_comb_body(hdim, with_cnt, mode),
        grid=(NB,),
        in_specs=[
            pl.BlockSpec((BLK, w), map_a),
            pl.BlockSpec((BLK, w), map_b),
            pl.BlockSpec((BLK, 1), lambda i: (i, 0)),
            pl.BlockSpec((1, hdim), lambda i: (0, 0)),
            pl.BlockSpec((BLK, 1), lambda i: (i, 0)),
        ],
        out_specs=out_specs,
        out_shape=out_shape,
    )(sa, sb, dinv, bvec, batch_f)


def _norm_relu(h, bf, S1, S2, S0, gw, gb, gms):
    cnt = jnp.maximum(S0[:, :1], 1.0)
    mean = S1 / cnt
    var = S2 / cnt + mean * mean * gms * (gms - 2.0)
    istd = lax.rsqrt(var + 1e-5)
    A = _onehot(bf, BLK)
    meanb = jnp.dot(A, gms * mean, preferred_element_type=F32, precision=lax.Precision.HIGHEST)
    istdb = jnp.dot(A, istd, preferred_element_type=F32, precision=lax.Precision.HIGHEST)
    hn = (h - meanb) * istdb * gw + gb
    return jnp.maximum(hn, 0.0)


def _c1_body(h_ref, bf_ref, S1_ref, S2_ref, S0_ref, gw_ref, gb_ref, gms_ref,
             dinv_ref, w_ref, y_ref):
    hr = _norm_relu(h_ref[...], bf_ref[...], S1_ref[...], S2_ref[...],
                    S0_ref[...], gw_ref[...], gb_ref[...], gms_ref[...])
    y_ref[...] = jnp.dot(hr * dinv_ref[...], w_ref[...],
                         preferred_element_type=F32, precision=lax.Precision.HIGHEST)


def _run_c1(h1, batch_f, S1, S2, S0, gw, gb, gms, dinv, W2):
    return pl.pallas_call(
        _c1_body,
        grid=(NB,),
        in_specs=[
            pl.BlockSpec((BLK, 256), lambda i: (i, 0)),
            pl.BlockSpec((BLK, 1), lambda i: (i, 0)),
            pl.BlockSpec((G, 256), lambda i: (0, 0)),
            pl.BlockSpec((G, 256), lambda i: (0, 0)),
            pl.BlockSpec((G, 128), lambda i: (0, 0)),
            pl.BlockSpec((1, 256), lambda i: (0, 0)),
            pl.BlockSpec((1, 256), lambda i: (0, 0)),
            pl.BlockSpec((1, 256), lambda i: (0, 0)),
            pl.BlockSpec((BLK, 1), lambda i: (i, 0)),
            pl.BlockSpec((256, 128), lambda i: (0, 0)),
        ],
        out_specs=pl.BlockSpec((BLK, 128), lambda i: (i, 0)),
        out_shape=jax.ShapeDtypeStruct((N, 128), F32),
    )(h1, batch_f, S1, S2, S0, gw, gb, gms, dinv, W2)


def _f_body(h_ref, bf_ref, S1_ref, S2_ref, S0_ref, gw_ref, gb_ref, gms_ref,
            w_ref, fb_ref, y_ref):
    hr = _norm_relu(h_ref[...], bf_ref[...], S1_ref[...], S2_ref[...],
                    S0_ref[...], gw_ref[...], gb_ref[...], gms_ref[...])
    y_ref[...] = jnp.dot(hr, w_ref[...], preferred_element_type=F32, precision=lax.Precision.HIGHEST) + fb_ref[...]


def _run_f(h2, batch_f, S1, S2, S0, gw, gb, gms, fcw8, fcb8):
    return pl.pallas_call(
        _f_body,
        grid=(NB,),
        in_specs=[
            pl.BlockSpec((BLK, 128), lambda i: (i, 0)),
            pl.BlockSpec((BLK, 1), lambda i: (i, 0)),
            pl.BlockSpec((G, 128), lambda i: (0, 0)),
            pl.BlockSpec((G, 128), lambda i: (0, 0)),
            pl.BlockSpec((G, 128), lambda i: (0, 0)),
            pl.BlockSpec((1, 128), lambda i: (0, 0)),
            pl.BlockSpec((1, 128), lambda i: (0, 0)),
            pl.BlockSpec((1, 128), lambda i: (0, 0)),
            pl.BlockSpec((128, 8), lambda i: (0, 0)),
            pl.BlockSpec((1, 8), lambda i: (0, 0)),
        ],
        out_specs=pl.BlockSpec((BLK, 8), lambda i: (i, 0)),
        out_shape=jax.ShapeDtypeStruct((N, 8), F32),
    )(h2, batch_f, S1, S2, S0, gw, gb, gms, fcw8, fcb8)


def kernel(x, index, batch, W1, b1, gn1_w, gn1_b, gn1_ms, W2, b2,
           gn2_w, gn2_b, gn2_ms, fc_W, fc_b):
    src = index[0]
    dst = index[1]
    batch_f = batch.astype(F32).reshape(N, 1)

    deg_a, deg_b = _deg_call(dst)
    y1, dinv = _run_a1(x, W1, deg_a.reshape(N, 1), deg_b.reshape(N, 1))
    s1 = _scat128(y1, src, dst)
    h1, S1, S2, S0 = _run_comb(s1, s1, dinv, b1.reshape(1, 256), batch_f,
                               256, True, "cat")
    y2 = _run_c1(h1, batch_f, S1, S2, S0, gn1_w.reshape(1, 256),
                 gn1_b.reshape(1, 256), gn1_ms.reshape(1, 256), dinv, W2)
    s2a, s2b = _scat_edge(y2, src, dst)
    h2, T1, T2 = _run_comb(s2a, s2b, dinv, b2.reshape(1, 128), batch_f,
                           128, False, "add")
    fcw8 = jnp.zeros((128, 8), F32).at[:, :2].set(fc_W)
    fcb8 = jnp.zeros((1, 8), F32).at[0, :2].set(fc_b)
    out8 = _run_f(h2, batch_f, T1, T2, S0, gn2_w.reshape(1, 128),
                  gn2_b.reshape(1, 128), gn2_ms.reshape(1, 128), fcw8, fcb8)
    return out8[:, :2]


# speedup vs baseline: 1.3040x; 1.3040x over previous
import functools

import jax
import jax.numpy as jnp
from jax import lax
from jax.experimental import pallas as pl
from jax.experimental.pallas import tpu as pltpu
from jax.experimental.pallas import tpu_sc as plsc

N = 10000
E = 320000
G = 64
NB = 10
BLK = 1000
C = 80
NSUB = 16
F32 = jnp.float32

@functools.lru_cache(maxsize=None)
def _mesh():
    return plsc.VectorSubcoreMesh(core_axis_name="c", subcore_axis_name="s")


def _deg_body(dst_hbm, deg_a, deg_b, dst_v, ones_v, zbuf, acc):
    cid = lax.axis_index("c")
    sid = lax.axis_index("s")
    for j in range(C // 16):
        ones_v[pl.ds(j * 16, 16)] = jnp.ones((16,), F32)
    for j in range(640 // 16):
        zbuf[pl.ds(j * 16, 16)] = jnp.zeros((16,), F32)

    @pl.when(sid < 15)
    def _():
        pltpu.sync_copy(zbuf, acc.at[pl.ds(sid * 640, 640)])

    @pl.when(sid == 15)
    def _():
        pltpu.sync_copy(zbuf.at[pl.ds(0, 400)], acc.at[pl.ds(9600, 400)])

    plsc.subcore_barrier()

    def step(k, carry):
        base = cid * (E // 2) + sid * (E // 2 // NSUB) + k * C
        pltpu.sync_copy(dst_hbm.at[pl.ds(base, C)], dst_v)
        pltpu.sync_copy(ones_v, acc.at[dst_v], add=True)
        return carry

    lax.fori_loop(0, E // 2 // NSUB // C, step, 0)
    plsc.subcore_barrier()

    def dump(out_ref, n):
        pltpu.sync_copy(acc.at[pl.ds(sid * 640, n)], zbuf.at[pl.ds(0, n)])
        pltpu.sync_copy(zbuf.at[pl.ds(0, n)], out_ref.at[pl.ds(sid * 640, n)])

    @pl.when(cid == 0)
    def _():
        @pl.when(sid < 15)
        def _():
            dump(deg_a, 640)

        @pl.when(sid == 15)
        def _():
            dump(deg_a, 400)

    @pl.when(cid == 1)
    def _():
        @pl.when(sid < 15)
        def _():
            dump(deg_b, 640)

        @pl.when(sid == 15)
        def _():
            dump(deg_b, 400)


def _deg_call(dst):
    return pl.kernel(
        _deg_body,
        out_type=[jax.ShapeDtypeStruct((N,), F32),
                  jax.ShapeDtypeStruct((N,), F32)],
        mesh=_mesh(),
        scratch_types=[
            pltpu.VMEM((C,), jnp.int32),
            pltpu.VMEM((C,), F32),
            pltpu.VMEM((640,), F32),
            pltpu.VMEM_SHARED((N,), F32),
        ],
    )(dst)


NRING = 4


def _edge_pipeline(y_hbm, idx_hbm, acc, bufs, yoff, ebase, nchunks, do_off):
    (src_hbm, dst_hbm) = idx_hbm
    (src_v, dst_v, rows_v, isem, gsem, ssem) = bufs
    if do_off:
        off = jnp.zeros((16,), jnp.int32) + yoff

    def start_idx(p, k):
        base = ebase + k * C
        pltpu.async_copy(src_hbm.at[pl.ds(base, C)], src_v[p], isem[p])
        pltpu.async_copy(dst_hbm.at[pl.ds(base, C)], dst_v[p], isem[p])

    def launch_gather(p, k):
        base = ebase + k * C
        pltpu.make_async_copy(src_hbm.at[pl.ds(base, C)], src_v[p],
                              isem[p]).wait()
        pltpu.make_async_copy(dst_hbm.at[pl.ds(base, C)], dst_v[p],
                              isem[p]).wait()
        if do_off:
            for j in range(C // 16):
                src_v[p][pl.ds(j * 16, 16)] = src_v[p][pl.ds(j * 16, 16)] + off
        pltpu.async_copy(y_hbm.at[src_v[p]], rows_v[p], gsem[p])

    def wait_gather(p):
        pltpu.make_async_copy(y_hbm.at[src_v[p]], rows_v[p], gsem[p]).wait()

    def start_scatter(p):
        pltpu.async_copy(rows_v[p], acc.at[dst_v[p]], ssem[p], add=True)

    def wait_scatter(p):
        pltpu.make_async_copy(rows_v[p], acc.at[dst_v[p]], ssem[p]).wait()

    ngroups, rem = divmod(nchunks, NRING)
    assert ngroups >= 1

    start_idx(0, 0)

    def substep(k, p, t):
        p1 = (p + 1) % NRING
        if p < NRING - 1:
            @pl.when(t >= 1)
            def _():
                wait_scatter(p1)
        else:
            wait_scatter(p1)

        @pl.when(k + 1 < nchunks)
        def _():
            start_idx(p1, k + 1)

        launch_gather(p, k)
        q = (p - 1) % NRING
        if p == 0:
            @pl.when(t >= 1)
            def _():
                wait_gather(q)
                start_scatter(q)
        else:
            wait_gather(q)
            start_scatter(q)

    def group(t, carry):
        for p in range(NRING):
            substep(NRING * t + p, p, t)
        return carry

    lax.fori_loop(0, ngroups, group, 0)
    for r in range(rem):
        k = ngroups * NRING + r
        p1 = (r + 1) % NRING
        wait_scatter(p1)
        if r + 1 < rem:
            start_idx(p1, k + 1)
        launch_gather(r, k)
        q = (r - 1) % NRING
        wait_gather(q)
        start_scatter(q)
    p_last = (nchunks - 1) % NRING
    wait_gather(p_last)
    start_scatter(p_last)
    for d in (3, 2, 1):
        wait_scatter((nchunks - d) % NRING)


def _stage_rows(nch, inner):
    def body(t, carry):
        inner(t)
        return carry

    lax.fori_loop(0, nch, body, 0)


def _make_scat(dh):
    def body(y_hbm, src_hbm, dst_hbm, out_hbm, *scr):
        src_v, dst_v, rows_v = scr[0:4], scr[4:8], scr[8:12]
        acc = scr[12]
        isem, gsem, ssem = scr[13:17], scr[17:21], scr[21:25]
        rows_a = rows_v[0]
        cid = lax.axis_index("c")
        sid = lax.axis_index("s")
        yoff = cid * N

        nch = jnp.where(sid == 15, 5, 8)

        def icopy(t):
            r0 = sid * 640 + t * C
            pltpu.sync_copy(y_hbm.at[pl.ds(yoff + r0, C)], rows_a)
            pltpu.sync_copy(rows_a, acc.at[pl.ds(r0, C)])

        _stage_rows(nch, icopy)
        plsc.subcore_barrier()

        bufs = (src_v, dst_v, rows_v, isem, gsem, ssem)
        _edge_pipeline(y_hbm, (src_hbm, dst_hbm), acc, bufs, yoff,
                       sid * (E // NSUB), E // NSUB // C, True)
        plsc.subcore_barrier()

        def ocopy(t):
            r0 = sid * 640 + t * C
            pltpu.sync_copy(acc.at[pl.ds(r0, C)], rows_a)
            pltpu.sync_copy(rows_a, out_hbm.at[pl.ds(yoff + r0, C)])

        _stage_rows(nch, ocopy)

    def run(y, src, dst):
        return pl.kernel(
            body,
            out_type=jax.ShapeDtypeStruct((2 * N, dh), F32),
            mesh=_mesh(),
            scratch_types=(
                [pltpu.VMEM((C,), jnp.int32)] * (2 * NRING)
                + [pltpu.VMEM((C, dh), F32)] * NRING
                + [pltpu.VMEM_SHARED((N, dh), F32)]
                + [pltpu.SemaphoreType.DMA] * (3 * NRING)
            ),
        )(y, src, dst)

    return run


_scat128 = _make_scat(128)


def _scat_edge_body(y_hbm, src_hbm, dst_hbm, out_a, out_b, *scr):
    src_v, dst_v, rows_v = scr[0:4], scr[4:8], scr[8:12]
    acc = scr[12]
    isem, gsem, ssem = scr[13:17], scr[17:21], scr[21:25]
    rows_a = rows_v[0]
    cid = lax.axis_index("c")
    sid = lax.axis_index("s")
    nch = jnp.where(sid == 15, 5, 8)

    @pl.when(cid == 1)
    def _():
        def zrow(r, carry):
            for j in range(128 // 16):
                rows_a[r, pl.ds(j * 16, 16)] = jnp.zeros((16,), F32)
            return carry

        lax.fori_loop(0, C, zrow, 0)

    def icopy(t):
        r0 = sid * 640 + t * C

        @pl.when(cid == 0)
        def _():
            pltpu.sync_copy(y_hbm.at[pl.ds(r0, C)], rows_a)

        pltpu.sync_copy(rows_a, acc.at[pl.ds(r0, C)])

    _stage_rows(nch, icopy)
    plsc.subcore_barrier()

    bufs = (src_v, dst_v, rows_v, isem, gsem, ssem)
    _edge_pipeline(y_hbm, (src_hbm, dst_hbm), acc, bufs, 0,
                   cid * (E // 2) + sid * (E // 2 // NSUB),
                   E // 2 // NSUB // C, False)
    plsc.subcore_barrier()

    def dump(out_ref):
        def ocopy(t):
            r0 = sid * 640 + t * C
            pltpu.sync_copy(acc.at[pl.ds(r0, C)], rows_a)
            pltpu.sync_copy(rows_a, out_ref.at[pl.ds(r0, C)])

        _stage_rows(nch, ocopy)

    @pl.when(cid == 0)
    def _():
        dump(out_a)

    @pl.when(cid == 1)
    def _():
        dump(out_b)


def _scat_edge(y, src, dst):
    return pl.kernel(
        _scat_edge_body,
        out_type=[jax.ShapeDtypeStruct((N, 128), F32),
                  jax.ShapeDtypeStruct((N, 128), F32)],
        mesh=_mesh(),
        scratch_types=(
            [pltpu.VMEM((C,), jnp.int32)] * (2 * NRING)
            + [pltpu.VMEM((C, 128), F32)] * NRING
            + [pltpu.VMEM_SHARED((N, 128), F32)]
            + [pltpu.SemaphoreType.DMA] * (3 * NRING)
        ),
    )(y, src, dst)


def _a1_body(x_ref, w_ref, da_ref, db_ref, y_ref, dinv_ref):
    dinv = lax.rsqrt(da_ref[...] + db_ref[...] + 1.0)
    dinv_ref[...] = dinv
    y_ref[...] = jnp.dot(x_ref[...] * dinv, w_ref[...],
                         preferred_element_type=F32, precision=lax.Precision.HIGHEST)


def _run_a1(x, W1, deg_a, deg_b):
    return pl.pallas_call(
        _a1_body,
        grid=(2, NB),
        in_specs=[
            pl.BlockSpec((BLK, 128), lambda h, i: (i, 0)),
            pl.BlockSpec((128, 128), lambda h, i: (0, h)),
            pl.BlockSpec((BLK, 1), lambda h, i: (i, 0)),
            pl.BlockSpec((BLK, 1), lambda h, i: (i, 0)),
        ],
        out_specs=[
            pl.BlockSpec((BLK, 128), lambda h, i: (h * NB + i, 0)),
            pl.BlockSpec((BLK, 1), lambda h, i: (i, 0)),
        ],
        out_shape=[
            jax.ShapeDtypeStruct((2 * N, 128), F32),
            jax.ShapeDtypeStruct((N, 1), F32),
        ],
    )(x, W1, deg_a, deg_b)


def _onehot(bcol, n_rows):
    iota = lax.broadcasted_iota(jnp.int32, (n_rows, G), 1).astype(F32)
    return (bcol == iota).astype(F32)


def _make_comb_body(hdim, with_cnt, mode):
    def body(s0_ref, s1_ref, dinv_ref, b_ref, bf_ref, h_ref, S1_ref, S2_ref,
             *maybe_S0):
        i = pl.program_id(0)
        if mode == "cat":
            s = jnp.concatenate([s0_ref[...], s1_ref[...]], axis=1)
        else:
            s = s0_ref[...] + s1_ref[...]
        h = s * dinv_ref[...] + b_ref[...]
        h_ref[...] = h
        A = _onehot(bf_ref[...], BLK)
        dn = (((0,), (0,)), ((), ()))
        p1 = lax.dot_general(A, h, dn, preferred_element_type=F32, precision=lax.Precision.HIGHEST)
        p2 = lax.dot_general(A, h * h, dn, preferred_element_type=F32, precision=lax.Precision.HIGHEST)

        @pl.when(i == 0)
        def _():
            S1_ref[...] = jnp.zeros((G, hdim), F32)
            S2_ref[...] = jnp.zeros((G, hdim), F32)
            if with_cnt:
                maybe_S0[0][...] = jnp.zeros((G, 128), F32)

        S1_ref[...] += p1
        S2_ref[...] += p2
        if with_cnt:
            p0 = lax.dot_general(A, jnp.ones((BLK, 128), F32), dn,
                                 preferred_element_type=F32, precision=lax.Precision.HIGHEST)
            maybe_S0[0][...] += p0

    return body


def _run_comb(sa, sb, dinv, bvec, batch_f, hdim, with_cnt, mode):
    if mode == "cat":
        w = hdim // 2
        map_a = lambda i: (i, 0)
        map_b = lambda i: (NB + i, 0)
    else:
        w = hdim
        map_a = lambda i: (i, 0)
        map_b = lambda i: (i, 0)
    out_shape = [
        jax.ShapeDtypeStruct((N, hdim), F32),
        jax.ShapeDtypeStruct((G, hdim), F32),
        jax.ShapeDtypeStruct((G, hdim), F32),
    ]
    out_specs = [
        pl.BlockSpec((BLK, hdim), lambda i: (i, 0)),
        pl.BlockSpec((G, hdim), lambda i: (0, 0)),
        pl.BlockSpec((G, hdim), lambda i: (0, 0)),
    ]
    if with_cnt:
        out_shape.append(jax.ShapeDtypeStruct((G, 128), F32))
        out_specs.append(pl.BlockSpec((G, 128), lambda i: (0, 0)))
    return pl.pallas_call(
        _make_comb_body(hdim, with_cnt, mode),
        grid=(NB,),
        in_specs=[
            pl.BlockSpec((BLK, w), map_a),
            pl.BlockSpec((BLK, w), map_b),
            pl.BlockSpec((BLK, 1), lambda i: (i, 0)),
            pl.BlockSpec((1, hdim), lambda i: (0, 0)),
            pl.BlockSpec((BLK, 1), lambda i: (i, 0)),
        ],
        out_specs=out_specs,
        out_shape=out_shape,
    )(sa, sb, dinv, bvec, batch_f)


def _norm_relu(h, bf, S1, S2, S0, gw, gb, gms):
    cnt = jnp.maximum(S0[:, :1], 1.0)
    mean = S1 / cnt
    var = S2 / cnt + mean * mean * gms * (gms - 2.0)
    istd = lax.rsqrt(var + 1e-5)
    A = _onehot(bf, BLK)
    meanb = jnp.dot(A, gms * mean, preferred_element_type=F32, precision=lax.Precision.HIGHEST)
    istdb = jnp.dot(A, istd, preferred_element_type=F32, precision=lax.Precision.HIGHEST)
    hn = (h - meanb) * istdb * gw + gb
    return jnp.maximum(hn, 0.0)


def _c1_body(h_ref, bf_ref, S1_ref, S2_ref, S0_ref, gw_ref, gb_ref, gms_ref,
             dinv_ref, w_ref, y_ref):
    hr = _norm_relu(h_ref[...], bf_ref[...], S1_ref[...], S2_ref[...],
                    S0_ref[...], gw_ref[...], gb_ref[...], gms_ref[...])
    y_ref[...] = jnp.dot(hr * dinv_ref[...], w_ref[...],
                         preferred_element_type=F32, precision=lax.Precision.HIGHEST)


def _run_c1(h1, batch_f, S1, S2, S0, gw, gb, gms, dinv, W2):
    return pl.pallas_call(
        _c1_body,
        grid=(NB,),
        in_specs=[
            pl.BlockSpec((BLK, 256), lambda i: (i, 0)),
            pl.BlockSpec((BLK, 1), lambda i: (i, 0)),
            pl.BlockSpec((G, 256), lambda i: (0, 0)),
            pl.BlockSpec((G, 256), lambda i: (0, 0)),
            pl.BlockSpec((G, 128), lambda i: (0, 0)),
            pl.BlockSpec((1, 256), lambda i: (0, 0)),
            pl.BlockSpec((1, 256), lambda i: (0, 0)),
            pl.BlockSpec((1, 256), lambda i: (0, 0)),
            pl.BlockSpec((BLK, 1), lambda i: (i, 0)),
            pl.BlockSpec((256, 128), lambda i: (0, 0)),
        ],
        out_specs=pl.BlockSpec((BLK, 128), lambda i: (i, 0)),
        out_shape=jax.ShapeDtypeStruct((N, 128), F32),
    )(h1, batch_f, S1, S2, S0, gw, gb, gms, dinv, W2)


def _f_body(h_ref, bf_ref, S1_ref, S2_ref, S0_ref, gw_ref, gb_ref, gms_ref,
            w_ref, fb_ref, y_ref):
    hr = _norm_relu(h_ref[...], bf_ref[...], S1_ref[...], S2_ref[...],
                    S0_ref[...], gw_ref[...], gb_ref[...], gms_ref[...])
    y_ref[...] = jnp.dot(hr, w_ref[...], preferred_element_type=F32, precision=lax.Precision.HIGHEST) + fb_ref[...]


def _run_f(h2, batch_f, S1, S2, S0, gw, gb, gms, fcw8, fcb8):
    return pl.pallas_call(
        _f_body,
        grid=(NB,),
        in_specs=[
            pl.BlockSpec((BLK, 128), lambda i: (i, 0)),
            pl.BlockSpec((BLK, 1), lambda i: (i, 0)),
            pl.BlockSpec((G, 128), lambda i: (0, 0)),
            pl.BlockSpec((G, 128), lambda i: (0, 0)),
            pl.BlockSpec((G, 128), lambda i: (0, 0)),
            pl.BlockSpec((1, 128), lambda i: (0, 0)),
            pl.BlockSpec((1, 128), lambda i: (0, 0)),
            pl.BlockSpec((1, 128), lambda i: (0, 0)),
            pl.BlockSpec((128, 8), lambda i: (0, 0)),
            pl.BlockSpec((1, 8), lambda i: (0, 0)),
        ],
        out_specs=pl.BlockSpec((BLK, 8), lambda i: (i, 0)),
        out_shape=jax.ShapeDtypeStruct((N, 8), F32),
    )(h2, batch_f, S1, S2, S0, gw, gb, gms, fcw8, fcb8)


def kernel(x, index, batch, W1, b1, gn1_w, gn1_b, gn1_ms, W2, b2,
           gn2_w, gn2_b, gn2_ms, fc_W, fc_b):
    src = index[0]
    dst = index[1]
    batch_f = batch.astype(F32).reshape(N, 1)

    deg_a, deg_b = _deg_call(dst)
    y1, dinv = _run_a1(x, W1, deg_a.reshape(N, 1), deg_b.reshape(N, 1))
    s1 = _scat128(y1, src, dst)
    h1, S1, S2, S0 = _run_comb(s1, s1, dinv, b1.reshape(1, 256), batch_f,
                               256, True, "cat")
    y2 = _run_c1(h1, batch_f, S1, S2, S0, gn1_w.reshape(1, 256),
                 gn1_b.reshape(1, 256), gn1_ms.reshape(1, 256), dinv, W2)
    s2a, s2b = _scat_edge(y2, src, dst)
    h2, T1, T2 = _run_comb(s2a, s2b, dinv, b2.reshape(1, 128), batch_f,
                           128, False, "add")
    fcw8 = jnp.zeros((128, 8), F32).at[:, :2].set(fc_W)
    fcb8 = jnp.zeros((1, 8), F32).at[0, :2].set(fc_b)
    out8 = _run_f(h2, batch_f, T1, T2, S0, gn2_w.reshape(1, 128),
                  gn2_b.reshape(1, 128), gn2_ms.reshape(1, 128), fcw8, fcb8)
    return out8[:, :2]


# speedup vs baseline: 1.3287x; 1.0190x over previous
import functools

import jax
import jax.numpy as jnp
from jax import lax
from jax.experimental import pallas as pl
from jax.experimental.pallas import tpu as pltpu
from jax.experimental.pallas import tpu_sc as plsc

N = 10000
E = 320000
G = 64
NB = 10
BLK = 1000
C = 80
NSUB = 16
F32 = jnp.float32

@functools.lru_cache(maxsize=None)
def _mesh():
    return plsc.VectorSubcoreMesh(core_axis_name="c", subcore_axis_name="s")


def _deg_body(idx_hbm, deg_a, deg_b, dst_v, ones_v, zbuf, acc):
    cid = lax.axis_index("c")
    sid = lax.axis_index("s")
    for j in range(C // 16):
        ones_v[pl.ds(j * 16, 16)] = jnp.ones((16,), F32)
    for j in range(640 // 16):
        zbuf[pl.ds(j * 16, 16)] = jnp.zeros((16,), F32)

    @pl.when(sid < 15)
    def _():
        pltpu.sync_copy(zbuf, acc.at[pl.ds(sid * 640, 640)])

    @pl.when(sid == 15)
    def _():
        pltpu.sync_copy(zbuf.at[pl.ds(0, 400)], acc.at[pl.ds(9600, 400)])

    plsc.subcore_barrier()

    def step(k, carry):
        base = cid * (E // 2) + sid * (E // 2 // NSUB) + k * C
        pltpu.sync_copy(idx_hbm.at[pl.ds(E + base, C)], dst_v)
        pltpu.sync_copy(ones_v, acc.at[dst_v], add=True)
        return carry

    lax.fori_loop(0, E // 2 // NSUB // C, step, 0)
    plsc.subcore_barrier()

    def dump(out_ref, n):
        pltpu.sync_copy(acc.at[pl.ds(sid * 640, n)], zbuf.at[pl.ds(0, n)])
        pltpu.sync_copy(zbuf.at[pl.ds(0, n)], out_ref.at[pl.ds(sid * 640, n)])

    @pl.when(cid == 0)
    def _():
        @pl.when(sid < 15)
        def _():
            dump(deg_a, 640)

        @pl.when(sid == 15)
        def _():
            dump(deg_a, 400)

    @pl.when(cid == 1)
    def _():
        @pl.when(sid < 15)
        def _():
            dump(deg_b, 640)

        @pl.when(sid == 15)
        def _():
            dump(deg_b, 400)


def _deg_call(idx_flat):
    return pl.kernel(
        _deg_body,
        out_type=[jax.ShapeDtypeStruct((N,), F32),
                  jax.ShapeDtypeStruct((N,), F32)],
        mesh=_mesh(),
        scratch_types=[
            pltpu.VMEM((C,), jnp.int32),
            pltpu.VMEM((C,), F32),
            pltpu.VMEM((640,), F32),
            pltpu.VMEM_SHARED((N,), F32),
        ],
    )(idx_flat)


NRING = 4


def _edge_pipeline(y_hbm, idx_hbm, acc, bufs, yoff, ebase, nchunks, do_off):
    (src_v, dst_v, rows_v, isem, gsem, ssem) = bufs
    if do_off:
        off = jnp.zeros((16,), jnp.int32) + yoff

    def start_idx(p, k):
        base = ebase + k * C
        pltpu.async_copy(idx_hbm.at[pl.ds(base, C)], src_v[p], isem[p])
        pltpu.async_copy(idx_hbm.at[pl.ds(E + base, C)], dst_v[p], isem[p])

    def launch_gather(p, k):
        base = ebase + k * C
        pltpu.make_async_copy(idx_hbm.at[pl.ds(base, C)], src_v[p],
                              isem[p]).wait()
        pltpu.make_async_copy(idx_hbm.at[pl.ds(E + base, C)], dst_v[p],
                              isem[p]).wait()
        if do_off:
            for j in range(C // 16):
                src_v[p][pl.ds(j * 16, 16)] = src_v[p][pl.ds(j * 16, 16)] + off
        pltpu.async_copy(y_hbm.at[src_v[p]], rows_v[p], gsem[p])

    def wait_gather(p):
        pltpu.make_async_copy(y_hbm.at[src_v[p]], rows_v[p], gsem[p]).wait()

    def start_scatter(p):
        pltpu.async_copy(rows_v[p], acc.at[dst_v[p]], ssem[p], add=True)

    def wait_scatter(p):
        pltpu.make_async_copy(rows_v[p], acc.at[dst_v[p]], ssem[p]).wait()

    ngroups, rem = divmod(nchunks, NRING)
    assert ngroups >= 1

    start_idx(0, 0)

    def substep(k, p, t):
        p1 = (p + 1) % NRING
        if p < NRING - 1:
            @pl.when(t >= 1)
            def _():
                wait_scatter(p1)
        else:
            wait_scatter(p1)

        @pl.when(k + 1 < nchunks)
        def _():
            start_idx(p1, k + 1)

        launch_gather(p, k)
        q = (p - 1) % NRING
        if p == 0:
            @pl.when(t >= 1)
            def _():
                wait_gather(q)
                start_scatter(q)
        else:
            wait_gather(q)
            start_scatter(q)

    def group(t, carry):
        for p in range(NRING):
            substep(NRING * t + p, p, t)
        return carry

    lax.fori_loop(0, ngroups, group, 0)
    for r in range(rem):
        k = ngroups * NRING + r
        p1 = (r + 1) % NRING
        wait_scatter(p1)
        if r + 1 < rem:
            start_idx(p1, k + 1)
        launch_gather(r, k)
        q = (r - 1) % NRING
        wait_gather(q)
        start_scatter(q)
    p_last = (nchunks - 1) % NRING
    wait_gather(p_last)
    start_scatter(p_last)
    for d in (3, 2, 1):
        wait_scatter((nchunks - d) % NRING)


def _stage_rows(nch, inner):
    def body(t, carry):
        inner(t)
        return carry

    lax.fori_loop(0, nch, body, 0)


def _make_scat(dh):
    def body(y_hbm, idx_hbm, out_hbm, *scr):
        src_v, dst_v, rows_v = scr[0:4], scr[4:8], scr[8:12]
        acc = scr[12]
        isem, gsem, ssem = scr[13:17], scr[17:21], scr[21:25]
        rows_a = rows_v[0]
        cid = lax.axis_index("c")
        sid = lax.axis_index("s")
        yoff = cid * N

        nch = jnp.where(sid == 15, 5, 8)

        def icopy(t):
            r0 = sid * 640 + t * C
            pltpu.sync_copy(y_hbm.at[pl.ds(yoff + r0, C)], rows_a)
            pltpu.sync_copy(rows_a, acc.at[pl.ds(r0, C)])

        _stage_rows(nch, icopy)
        plsc.subcore_barrier()

        bufs = (src_v, dst_v, rows_v, isem, gsem, ssem)
        _edge_pipeline(y_hbm, idx_hbm, acc, bufs, yoff,
                       sid * (E // NSUB), E // NSUB // C, True)
        plsc.subcore_barrier()

        def ocopy(t):
            r0 = sid * 640 + t * C
            pltpu.sync_copy(acc.at[pl.ds(r0, C)], rows_a)
            pltpu.sync_copy(rows_a, out_hbm.at[pl.ds(yoff + r0, C)])

        _stage_rows(nch, ocopy)

    def run(y, idx_flat):
        return pl.kernel(
            body,
            out_type=jax.ShapeDtypeStruct((2 * N, dh), F32),
            mesh=_mesh(),
            scratch_types=(
                [pltpu.VMEM((C,), jnp.int32)] * (2 * NRING)
                + [pltpu.VMEM((C, dh), F32)] * NRING
                + [pltpu.VMEM_SHARED((N, dh), F32)]
                + [pltpu.SemaphoreType.DMA] * (3 * NRING)
            ),
        )(y, idx_flat)

    return run


_scat128 = _make_scat(128)


def _scat_edge_body(y_hbm, idx_hbm, out_a, out_b, *scr):
    src_v, dst_v, rows_v = scr[0:4], scr[4:8], scr[8:12]
    acc = scr[12]
    isem, gsem, ssem = scr[13:17], scr[17:21], scr[21:25]
    rows_a = rows_v[0]
    cid = lax.axis_index("c")
    sid = lax.axis_index("s")
    nch = jnp.where(sid == 15, 5, 8)

    @pl.when(cid == 1)
    def _():
        def zrow(r, carry):
            for j in range(128 // 16):
                rows_a[r, pl.ds(j * 16, 16)] = jnp.zeros((16,), F32)
            return carry

        lax.fori_loop(0, C, zrow, 0)

    def icopy(t):
        r0 = sid * 640 + t * C

        @pl.when(cid == 0)
        def _():
            pltpu.sync_copy(y_hbm.at[pl.ds(r0, C)], rows_a)

        pltpu.sync_copy(rows_a, acc.at[pl.ds(r0, C)])

    _stage_rows(nch, icopy)
    plsc.subcore_barrier()

    bufs = (src_v, dst_v, rows_v, isem, gsem, ssem)
    _edge_pipeline(y_hbm, idx_hbm, acc, bufs, 0,
                   cid * (E // 2) + sid * (E // 2 // NSUB),
                   E // 2 // NSUB // C, False)
    plsc.subcore_barrier()

    def dump(out_ref):
        def ocopy(t):
            r0 = sid * 640 + t * C
            pltpu.sync_copy(acc.at[pl.ds(r0, C)], rows_a)
            pltpu.sync_copy(rows_a, out_ref.at[pl.ds(r0, C)])

        _stage_rows(nch, ocopy)

    @pl.when(cid == 0)
    def _():
        dump(out_a)

    @pl.when(cid == 1)
    def _():
        dump(out_b)


def _scat_edge(y, idx_flat):
    return pl.kernel(
        _scat_edge_body,
        out_type=[jax.ShapeDtypeStruct((N, 128), F32),
                  jax.ShapeDtypeStruct((N, 128), F32)],
        mesh=_mesh(),
        scratch_types=(
            [pltpu.VMEM((C,), jnp.int32)] * (2 * NRING)
            + [pltpu.VMEM((C, 128), F32)] * NRING
            + [pltpu.VMEM_SHARED((N, 128), F32)]
            + [pltpu.SemaphoreType.DMA] * (3 * NRING)
        ),
    )(y, idx_flat)


def _a1_body(x_ref, w_ref, da_ref, db_ref, y_ref, dinv_ref):
    dinv = lax.rsqrt(da_ref[...] + db_ref[...] + 1.0)
    dinv_ref[...] = dinv
    y_ref[...] = jnp.dot(x_ref[...] * dinv, w_ref[...],
                         preferred_element_type=F32, precision=lax.Precision.HIGHEST)


def _run_a1(x, W1, deg_a, deg_b):
    return pl.pallas_call(
        _a1_body,
        grid=(2, NB),
        in_specs=[
            pl.BlockSpec((BLK, 128), lambda h, i: (i, 0)),
            pl.BlockSpec((128, 128), lambda h, i: (0, h)),
            pl.BlockSpec((BLK, 1), lambda h, i: (i, 0)),
            pl.BlockSpec((BLK, 1), lambda h, i: (i, 0)),
        ],
        out_specs=[
            pl.BlockSpec((BLK, 128), lambda h, i: (h * NB + i, 0)),
            pl.BlockSpec((BLK, 1), lambda h, i: (i, 0)),
        ],
        out_shape=[
            jax.ShapeDtypeStruct((2 * N, 128), F32),
            jax.ShapeDtypeStruct((N, 1), F32),
        ],
    )(x, W1, deg_a, deg_b)


def _onehot(bcol, n_rows):
    iota = lax.broadcasted_iota(jnp.int32, (n_rows, G), 1).astype(F32)
    return (bcol == iota).astype(F32)


def _make_comb_body(hdim, with_cnt, mode):
    def body(s0_ref, s1_ref, dinv_ref, b_ref, bf_ref, h_ref, S1_ref, S2_ref,
             *maybe_S0):
        i = pl.program_id(0)
        if mode == "cat":
            s = jnp.concatenate([s0_ref[...], s1_ref[...]], axis=1)
        else:
            s = s0_ref[...] + s1_ref[...]
        h = s * dinv_ref[...] + b_ref[...]
        h_ref[...] = h
        A = _onehot(bf_ref[...], BLK)
        dn = (((0,), (0,)), ((), ()))
        p1 = lax.dot_general(A, h, dn, preferred_element_type=F32, precision=lax.Precision.HIGHEST)
        p2 = lax.dot_general(A, h * h, dn, preferred_element_type=F32, precision=lax.Precision.HIGHEST)

        @pl.when(i == 0)
        def _():
            S1_ref[...] = jnp.zeros((G, hdim), F32)
            S2_ref[...] = jnp.zeros((G, hdim), F32)
            if with_cnt:
                maybe_S0[0][...] = jnp.zeros((G, 128), F32)

        S1_ref[...] += p1
        S2_ref[...] += p2
        if with_cnt:
            p0 = lax.dot_general(A, jnp.ones((BLK, 128), F32), dn,
                                 preferred_element_type=F32, precision=lax.Precision.HIGHEST)
            maybe_S0[0][...] += p0

    return body


def _run_comb(sa, sb, dinv, bvec, batch_f, hdim, with_cnt, mode):
    if mode == "cat":
        w = hdim // 2
        map_a = lambda i: (i, 0)
        map_b = lambda i: (NB + i, 0)
    else:
        w = hdim
        map_a = lambda i: (i, 0)
        map_b = lambda i: (i, 0)
    out_shape = [
        jax.ShapeDtypeStruct((N, hdim), F32),
        jax.ShapeDtypeStruct((G, hdim), F32),
        jax.ShapeDtypeStruct((G, hdim), F32),
    ]
    out_specs = [
        pl.BlockSpec((BLK, hdim), lambda i: (i, 0)),
        pl.BlockSpec((G, hdim), lambda i: (0, 0)),
        pl.BlockSpec((G, hdim), lambda i: (0, 0)),
    ]
    if with_cnt:
        out_shape.append(jax.ShapeDtypeStruct((G, 128), F32))
        out_specs.append(pl.BlockSpec((G, 128), lambda i: (0, 0)))
    return pl.pallas_call(
        _make_comb_body(hdim, with_cnt, mode),
        grid=(NB,),
        in_specs=[
            pl.BlockSpec((BLK, w), map_a),
            pl.BlockSpec((BLK, w), map_b),
            pl.BlockSpec((BLK, 1), lambda i: (i, 0)),
            pl.BlockSpec((1, hdim), lambda i: (0, 0)),
            pl.BlockSpec((BLK, 1), lambda i: (i, 0)),
        ],
        out_specs=out_specs,
        out_shape=out_shape,
    )(sa, sb, dinv, bvec, batch_f)


def _norm_relu(h, bf, S1, S2, S0, gw, gb, gms):
    cnt = jnp.maximum(S0[:, :1], 1.0)
    mean = S1 / cnt
    var = S2 / cnt + mean * mean * gms * (gms - 2.0)
    istd = lax.rsqrt(var + 1e-5)
    A = _onehot(bf, BLK)
    meanb = jnp.dot(A, gms * mean, preferred_element_type=F32, precision=lax.Precision.HIGHEST)
    istdb = jnp.dot(A, istd, preferred_element_type=F32, precision=lax.Precision.HIGHEST)
    hn = (h - meanb) * istdb * gw + gb
    return jnp.maximum(hn, 0.0)


def _c1_body(h_ref, bf_ref, S1_ref, S2_ref, S0_ref, gw_ref, gb_ref, gms_ref,
             dinv_ref, w_ref, y_ref):
    hr = _norm_relu(h_ref[...], bf_ref[...], S1_ref[...], S2_ref[...],
                    S0_ref[...], gw_ref[...], gb_ref[...], gms_ref[...])
    y_ref[...] = jnp.dot(hr * dinv_ref[...], w_ref[...],
                         preferred_element_type=F32, precision=lax.Precision.HIGHEST)


def _run_c1(h1, batch_f, S1, S2, S0, gw, gb, gms, dinv, W2):
    return pl.pallas_call(
        _c1_body,
        grid=(NB,),
        in_specs=[
            pl.BlockSpec((BLK, 256), lambda i: (i, 0)),
            pl.BlockSpec((BLK, 1), lambda i: (i, 0)),
            pl.BlockSpec((G, 256), lambda i: (0, 0)),
            pl.BlockSpec((G, 256), lambda i: (0, 0)),
            pl.BlockSpec((G, 128), lambda i: (0, 0)),
            pl.BlockSpec((1, 256), lambda i: (0, 0)),
            pl.BlockSpec((1, 256), lambda i: (0, 0)),
            pl.BlockSpec((1, 256), lambda i: (0, 0)),
            pl.BlockSpec((BLK, 1), lambda i: (i, 0)),
            pl.BlockSpec((256, 128), lambda i: (0, 0)),
        ],
        out_specs=pl.BlockSpec((BLK, 128), lambda i: (i, 0)),
        out_shape=jax.ShapeDtypeStruct((N, 128), F32),
    )(h1, batch_f, S1, S2, S0, gw, gb, gms, dinv, W2)


def _f_body(h_ref, bf_ref, S1_ref, S2_ref, S0_ref, gw_ref, gb_ref, gms_ref,
            w_ref, fb_ref, y_ref):
    hr = _norm_relu(h_ref[...], bf_ref[...], S1_ref[...], S2_ref[...],
                    S0_ref[...], gw_ref[...], gb_ref[...], gms_ref[...])
    y_ref[...] = jnp.dot(hr, w_ref[...], preferred_element_type=F32, precision=lax.Precision.HIGHEST) + fb_ref[...]


def _run_f(h2, batch_f, S1, S2, S0, gw, gb, gms, fcw8, fcb8):
    return pl.pallas_call(
        _f_body,
        grid=(NB,),
        in_specs=[
            pl.BlockSpec((BLK, 128), lambda i: (i, 0)),
            pl.BlockSpec((BLK, 1), lambda i: (i, 0)),
            pl.BlockSpec((G, 128), lambda i: (0, 0)),
            pl.BlockSpec((G, 128), lambda i: (0, 0)),
            pl.BlockSpec((G, 128), lambda i: (0, 0)),
            pl.BlockSpec((1, 128), lambda i: (0, 0)),
            pl.BlockSpec((1, 128), lambda i: (0, 0)),
            pl.BlockSpec((1, 128), lambda i: (0, 0)),
            pl.BlockSpec((128, 8), lambda i: (0, 0)),
            pl.BlockSpec((1, 8), lambda i: (0, 0)),
        ],
        out_specs=pl.BlockSpec((BLK, 8), lambda i: (i, 0)),
        out_shape=jax.ShapeDtypeStruct((N, 8), F32),
    )(h2, batch_f, S1, S2, S0, gw, gb, gms, fcw8, fcb8)


def kernel(x, index, batch, W1, b1, gn1_w, gn1_b, gn1_ms, W2, b2,
           gn2_w, gn2_b, gn2_ms, fc_W, fc_b):
    idx_flat = index.reshape(2 * E)
    batch_f = batch.astype(F32).reshape(N, 1)

    deg_a, deg_b = _deg_call(idx_flat)
    y1, dinv = _run_a1(x, W1, deg_a[:N].reshape(N, 1), deg_b[:N].reshape(N, 1))
    s1 = _scat128(y1, idx_flat)
    h1, S1, S2, S0 = _run_comb(s1, s1, dinv, b1.reshape(1, 256), batch_f,
                               256, True, "cat")
    y2 = _run_c1(h1, batch_f, S1, S2, S0, gn1_w.reshape(1, 256),
                 gn1_b.reshape(1, 256), gn1_ms.reshape(1, 256), dinv, W2)
    s2a, s2b = _scat_edge(y2, idx_flat)
    h2, T1, T2 = _run_comb(s2a, s2b, dinv, b2.reshape(1, 128), batch_f,
                           128, False, "add")
    fcw8 = jnp.zeros((128, 8), F32).at[:, :2].set(fc_W)
    fcb8 = jnp.zeros((1, 8), F32).at[0, :2].set(fc_b)
    out8 = _run_f(h2, batch_f, T1, T2, S0, gn2_w.reshape(1, 128),
                  gn2_b.reshape(1, 128), gn2_ms.reshape(1, 128), fcw8, fcb8)
    return out8[:, :2]


# speedup vs baseline: 1.3389x; 1.0076x over previous
import functools

import jax
import jax.numpy as jnp
from jax import lax
from jax.experimental import pallas as pl
from jax.experimental.pallas import tpu as pltpu
from jax.experimental.pallas import tpu_sc as plsc

N = 10000
E = 320000
G = 64
NB = 10
BLK = 1000
C = 80
NSUB = 16
F32 = jnp.float32

@functools.lru_cache(maxsize=None)
def _mesh():
    return plsc.VectorSubcoreMesh(core_axis_name="c", subcore_axis_name="s")


def _deg_body(idx_hbm, deg_a, deg_b, dst_v, ones_v, zbuf, acc):
    cid = lax.axis_index("c")
    sid = lax.axis_index("s")
    for j in range(C // 16):
        ones_v[pl.ds(j * 16, 16)] = jnp.ones((16,), F32)
    for j in range(640 // 16):
        zbuf[pl.ds(j * 16, 16)] = jnp.zeros((16,), F32)

    @pl.when(sid < 15)
    def _():
        pltpu.sync_copy(zbuf, acc.at[pl.ds(sid * 640, 640)])

    @pl.when(sid == 15)
    def _():
        pltpu.sync_copy(zbuf.at[pl.ds(0, 400)], acc.at[pl.ds(9600, 400)])

    plsc.subcore_barrier()

    def step(k, carry):
        base = cid * (E // 2) + sid * (E // 2 // NSUB) + k * C
        pltpu.sync_copy(idx_hbm.at[pl.ds(E + base, C)], dst_v)
        pltpu.sync_copy(ones_v, acc.at[dst_v], add=True)
        return carry

    lax.fori_loop(0, E // 2 // NSUB // C, step, 0)
    plsc.subcore_barrier()

    def dump(out_ref, n):
        pltpu.sync_copy(acc.at[pl.ds(sid * 640, n)], zbuf.at[pl.ds(0, n)])
        pltpu.sync_copy(zbuf.at[pl.ds(0, n)], out_ref.at[pl.ds(sid * 640, n)])

    @pl.when(cid == 0)
    def _():
        @pl.when(sid < 15)
        def _():
            dump(deg_a, 640)

        @pl.when(sid == 15)
        def _():
            dump(deg_a, 400)

    @pl.when(cid == 1)
    def _():
        @pl.when(sid < 15)
        def _():
            dump(deg_b, 640)

        @pl.when(sid == 15)
        def _():
            dump(deg_b, 400)


def _deg_call(idx_flat):
    return pl.kernel(
        _deg_body,
        out_type=[jax.ShapeDtypeStruct((N,), F32),
                  jax.ShapeDtypeStruct((N,), F32)],
        mesh=_mesh(),
        scratch_types=[
            pltpu.VMEM((C,), jnp.int32),
            pltpu.VMEM((C,), F32),
            pltpu.VMEM((640,), F32),
            pltpu.VMEM_SHARED((N,), F32),
        ],
    )(idx_flat)


NRING = 4


def _edge_pipeline(y_hbm, idx_hbm, acc, bufs, yoff, ebase, nchunks, do_off):
    (src_v, dst_v, rows_v, isem, gsem, ssem) = bufs
    if do_off:
        off = jnp.zeros((16,), jnp.int32) + yoff

    def start_idx(p, k):
        base = ebase + k * C
        pltpu.async_copy(idx_hbm.at[pl.ds(base, C)], src_v[p], isem[p])
        pltpu.async_copy(idx_hbm.at[pl.ds(E + base, C)], dst_v[p], isem[p])

    def launch_gather(p, k):
        base = ebase + k * C
        pltpu.make_async_copy(idx_hbm.at[pl.ds(base, C)], src_v[p],
                              isem[p]).wait()
        pltpu.make_async_copy(idx_hbm.at[pl.ds(E + base, C)], dst_v[p],
                              isem[p]).wait()
        if do_off:
            for j in range(C // 16):
                src_v[p][pl.ds(j * 16, 16)] = src_v[p][pl.ds(j * 16, 16)] + off
        pltpu.async_copy(y_hbm.at[src_v[p]], rows_v[p], gsem[p])

    def wait_gather(p):
        pltpu.make_async_copy(y_hbm.at[src_v[p]], rows_v[p], gsem[p]).wait()

    def start_scatter(p):
        pltpu.async_copy(rows_v[p], acc.at[dst_v[p]], ssem[p], add=True)

    def wait_scatter(p):
        pltpu.make_async_copy(rows_v[p], acc.at[dst_v[p]], ssem[p]).wait()

    ngroups, rem = divmod(nchunks, NRING)
    assert ngroups >= 1

    start_idx(0, 0)

    def substep(k, p, t):
        p1 = (p + 1) % NRING
        if p < NRING - 1:
            @pl.when(t >= 1)
            def _():
                wait_scatter(p1)
        else:
            wait_scatter(p1)

        @pl.when(k + 1 < nchunks)
        def _():
            start_idx(p1, k + 1)

        launch_gather(p, k)
        q = (p - 1) % NRING
        if p == 0:
            @pl.when(t >= 1)
            def _():
                wait_gather(q)
                start_scatter(q)
        else:
            wait_gather(q)
            start_scatter(q)

    def group(t, carry):
        for p in range(NRING):
            substep(NRING * t + p, p, t)
        return carry

    lax.fori_loop(0, ngroups, group, 0)
    for r in range(rem):
        k = ngroups * NRING + r
        p1 = (r + 1) % NRING
        wait_scatter(p1)
        if r + 1 < rem:
            start_idx(p1, k + 1)
        launch_gather(r, k)
        q = (r - 1) % NRING
        wait_gather(q)
        start_scatter(q)
    p_last = (nchunks - 1) % NRING
    wait_gather(p_last)
    start_scatter(p_last)
    for d in (3, 2, 1):
        wait_scatter((nchunks - d) % NRING)


def _stage_rows(nch, inner):
    def body(t, carry):
        inner(t)
        return carry

    lax.fori_loop(0, nch, body, 0)


def _make_scat(dh):
    def body(y_hbm, idx_hbm, out_hbm, *scr):
        src_v, dst_v, rows_v = scr[0:4], scr[4:8], scr[8:12]
        acc = scr[12]
        isem, gsem, ssem = scr[13:17], scr[17:21], scr[21:25]
        rows_a = rows_v[0]
        cid = lax.axis_index("c")
        sid = lax.axis_index("s")
        yoff = cid * N

        nch = jnp.where(sid == 15, 5, 8)

        def icopy(t):
            r0 = sid * 640 + t * C
            pltpu.sync_copy(y_hbm.at[pl.ds(yoff + r0, C)], rows_a)
            pltpu.sync_copy(rows_a, acc.at[pl.ds(r0, C)])

        _stage_rows(nch, icopy)
        plsc.subcore_barrier()

        bufs = (src_v, dst_v, rows_v, isem, gsem, ssem)
        _edge_pipeline(y_hbm, idx_hbm, acc, bufs, yoff,
                       sid * (E // NSUB), E // NSUB // C, True)
        plsc.subcore_barrier()

        def ocopy(t):
            r0 = sid * 640 + t * C
            pltpu.sync_copy(acc.at[pl.ds(r0, C)], rows_a)
            pltpu.sync_copy(rows_a, out_hbm.at[pl.ds(yoff + r0, C)])

        _stage_rows(nch, ocopy)

    def run(y, idx_flat):
        return pl.kernel(
            body,
            out_type=jax.ShapeDtypeStruct((2 * N, dh), F32),
            mesh=_mesh(),
            scratch_types=(
                [pltpu.VMEM((C,), jnp.int32)] * (2 * NRING)
                + [pltpu.VMEM((C, dh), F32)] * NRING
                + [pltpu.VMEM_SHARED((N, dh), F32)]
                + [pltpu.SemaphoreType.DMA] * (3 * NRING)
            ),
        )(y, idx_flat)

    return run


_scat128 = _make_scat(128)


def _scat_edge_body(y_hbm, idx_hbm, out_a, out_b, *scr):
    src_v, dst_v, rows_v = scr[0:4], scr[4:8], scr[8:12]
    acc = scr[12]
    isem, gsem, ssem = scr[13:17], scr[17:21], scr[21:25]
    rows_a = rows_v[0]
    cid = lax.axis_index("c")
    sid = lax.axis_index("s")
    nch = jnp.where(sid == 15, 5, 8)

    @pl.when(cid == 1)
    def _():
        def zrow(r, carry):
            for j in range(128 // 16):
                rows_a[r, pl.ds(j * 16, 16)] = jnp.zeros((16,), F32)
            return carry

        lax.fori_loop(0, C, zrow, 0)

    def icopy(t):
        r0 = sid * 640 + t * C

        @pl.when(cid == 0)
        def _():
            pltpu.sync_copy(y_hbm.at[pl.ds(r0, C)], rows_a)

        pltpu.sync_copy(rows_a, acc.at[pl.ds(r0, C)])

    _stage_rows(nch, icopy)
    plsc.subcore_barrier()

    bufs = (src_v, dst_v, rows_v, isem, gsem, ssem)
    _edge_pipeline(y_hbm, idx_hbm, acc, bufs, 0,
                   cid * (E // 2) + sid * (E // 2 // NSUB),
                   E // 2 // NSUB // C, False)
    plsc.subcore_barrier()

    def dump(out_ref):
        def ocopy(t):
            r0 = sid * 640 + t * C
            pltpu.sync_copy(acc.at[pl.ds(r0, C)], rows_a)
            pltpu.sync_copy(rows_a, out_ref.at[pl.ds(r0, C)])

        _stage_rows(nch, ocopy)

    @pl.when(cid == 0)
    def _():
        dump(out_a)

    @pl.when(cid == 1)
    def _():
        dump(out_b)


def _scat_edge(y, idx_flat):
    return pl.kernel(
        _scat_edge_body,
        out_type=[jax.ShapeDtypeStruct((N, 128), F32),
                  jax.ShapeDtypeStruct((N, 128), F32)],
        mesh=_mesh(),
        scratch_types=(
            [pltpu.VMEM((C,), jnp.int32)] * (2 * NRING)
            + [pltpu.VMEM((C, 128), F32)] * NRING
            + [pltpu.VMEM_SHARED((N, 128), F32)]
            + [pltpu.SemaphoreType.DMA] * (3 * NRING)
        ),
    )(y, idx_flat)


def _a1_body(x_ref, w_ref, da_ref, db_ref, y_ref, dinv_ref):
    dinv = lax.rsqrt(da_ref[...] + db_ref[...] + 1.0)
    dinv_ref[...] = dinv
    y_ref[...] = jnp.dot(x_ref[...] * dinv, w_ref[...],
                         preferred_element_type=F32, precision=lax.Precision.HIGHEST)


def _run_a1(x, W1, deg_a, deg_b):
    return pl.pallas_call(
        _a1_body,
        grid=(2, NB),
        in_specs=[
            pl.BlockSpec((BLK, 128), lambda h, i: (i, 0)),
            pl.BlockSpec((128, 128), lambda h, i: (0, h)),
            pl.BlockSpec((BLK, 1), lambda h, i: (i, 0)),
            pl.BlockSpec((BLK, 1), lambda h, i: (i, 0)),
        ],
        out_specs=[
            pl.BlockSpec((BLK, 128), lambda h, i: (h * NB + i, 0)),
            pl.BlockSpec((BLK, 1), lambda h, i: (i, 0)),
        ],
        out_shape=[
            jax.ShapeDtypeStruct((2 * N, 128), F32),
            jax.ShapeDtypeStruct((N, 1), F32),
        ],
    )(x, W1, deg_a, deg_b)


def _onehot(bcol, n_rows):
    iota = lax.broadcasted_iota(jnp.int32, (n_rows, G), 1).astype(F32)
    return (bcol == iota).astype(F32)


def _l1_fused_body(s0_ref, s1_ref, dinv_ref, b_ref, bf_ref, gw_ref, gb_ref,
                   gms_ref, w_ref, y_ref, S0_ref, hs, S1s, S2s):
    ph = pl.program_id(0)
    i = pl.program_id(1)
    dn = (((0,), (0,)), ((), ()))

    @pl.when(ph == 0)
    def _():
        h = (jnp.concatenate([s0_ref[...], s1_ref[...]], axis=1)
             * dinv_ref[...] + b_ref[...])
        hs[pl.ds(i * BLK, BLK), :] = h
        A = _onehot(bf_ref[...], BLK)
        p1 = lax.dot_general(A, h, dn, preferred_element_type=F32,
                             precision=lax.Precision.HIGHEST)
        p2 = lax.dot_general(A, h * h, dn, preferred_element_type=F32,
                             precision=lax.Precision.HIGHEST)
        p0 = lax.dot_general(A, jnp.ones((BLK, 128), F32), dn,
                             preferred_element_type=F32,
                             precision=lax.Precision.HIGHEST)

        @pl.when(i == 0)
        def _():
            S1s[...] = jnp.zeros((G, 256), F32)
            S2s[...] = jnp.zeros((G, 256), F32)
            S0_ref[...] = jnp.zeros((G, 128), F32)

        S1s[...] += p1
        S2s[...] += p2
        S0_ref[...] += p0

    @pl.when(ph == 1)
    def _():
        h = hs[pl.ds(i * BLK, BLK), :]
        hr = _norm_relu(h, bf_ref[...], S1s[...], S2s[...], S0_ref[...],
                        gw_ref[...], gb_ref[...], gms_ref[...])
        y_ref[...] = jnp.dot(hr * dinv_ref[...], w_ref[...],
                             preferred_element_type=F32,
                             precision=lax.Precision.HIGHEST)


def _run_l1(s1, dinv, bvec, batch_f, gw, gb, gms, W2):
    return pl.pallas_call(
        _l1_fused_body,
        grid=(2, NB),
        in_specs=[
            pl.BlockSpec((BLK, 128), lambda ph, i: (i, 0)),
            pl.BlockSpec((BLK, 128), lambda ph, i: (NB + i, 0)),
            pl.BlockSpec((BLK, 1), lambda ph, i: (i, 0)),
            pl.BlockSpec((1, 256), lambda ph, i: (0, 0)),
            pl.BlockSpec((BLK, 1), lambda ph, i: (i, 0)),
            pl.BlockSpec((1, 256), lambda ph, i: (0, 0)),
            pl.BlockSpec((1, 256), lambda ph, i: (0, 0)),
            pl.BlockSpec((1, 256), lambda ph, i: (0, 0)),
            pl.BlockSpec((256, 128), lambda ph, i: (0, 0)),
        ],
        out_specs=[
            pl.BlockSpec((BLK, 128), lambda ph, i: (i, 0)),
            pl.BlockSpec((G, 128), lambda ph, i: (0, 0)),
        ],
        out_shape=[
            jax.ShapeDtypeStruct((N, 128), F32),
            jax.ShapeDtypeStruct((G, 128), F32),
        ],
        scratch_shapes=[
            pltpu.VMEM((N, 256), F32),
            pltpu.VMEM((G, 256), F32),
            pltpu.VMEM((G, 256), F32),
        ],
    )(s1, s1, dinv, bvec, batch_f, gw, gb, gms, W2)


def _l2_fused_body(sa_ref, sb_ref, dinv_ref, b_ref, bf_ref, gw_ref, gb_ref,
                   gms_ref, S0_ref, w_ref, fb_ref, y_ref, hs, S1s, S2s):
    ph = pl.program_id(0)
    i = pl.program_id(1)
    dn = (((0,), (0,)), ((), ()))

    @pl.when(ph == 0)
    def _():
        h = (sa_ref[...] + sb_ref[...]) * dinv_ref[...] + b_ref[...]
        hs[pl.ds(i * BLK, BLK), :] = h
        A = _onehot(bf_ref[...], BLK)
        p1 = lax.dot_general(A, h, dn, preferred_element_type=F32,
                             precision=lax.Precision.HIGHEST)
        p2 = lax.dot_general(A, h * h, dn, preferred_element_type=F32,
                             precision=lax.Precision.HIGHEST)

        @pl.when(i == 0)
        def _():
            S1s[...] = jnp.zeros((G, 128), F32)
            S2s[...] = jnp.zeros((G, 128), F32)

        S1s[...] += p1
        S2s[...] += p2

    @pl.when(ph == 1)
    def _():
        h = hs[pl.ds(i * BLK, BLK), :]
        hr = _norm_relu(h, bf_ref[...], S1s[...], S2s[...], S0_ref[...],
                        gw_ref[...], gb_ref[...], gms_ref[...])
        y_ref[...] = jnp.dot(hr, w_ref[...], preferred_element_type=F32,
                             precision=lax.Precision.HIGHEST) + fb_ref[...]


def _run_l2(s2a, s2b, dinv, bvec, batch_f, gw, gb, gms, S0, fcw8, fcb8):
    return pl.pallas_call(
        _l2_fused_body,
        grid=(2, NB),
        in_specs=[
            pl.BlockSpec((BLK, 128), lambda ph, i: (i, 0)),
            pl.BlockSpec((BLK, 128), lambda ph, i: (i, 0)),
            pl.BlockSpec((BLK, 1), lambda ph, i: (i, 0)),
            pl.BlockSpec((1, 128), lambda ph, i: (0, 0)),
            pl.BlockSpec((BLK, 1), lambda ph, i: (i, 0)),
            pl.BlockSpec((1, 128), lambda ph, i: (0, 0)),
            pl.BlockSpec((1, 128), lambda ph, i: (0, 0)),
            pl.BlockSpec((1, 128), lambda ph, i: (0, 0)),
            pl.BlockSpec((G, 128), lambda ph, i: (0, 0)),
            pl.BlockSpec((128, 8), lambda ph, i: (0, 0)),
            pl.BlockSpec((1, 8), lambda ph, i: (0, 0)),
        ],
        out_specs=pl.BlockSpec((BLK, 8), lambda ph, i: (i, 0)),
        out_shape=jax.ShapeDtypeStruct((N, 8), F32),
        scratch_shapes=[
            pltpu.VMEM((N, 128), F32),
            pltpu.VMEM((G, 128), F32),
            pltpu.VMEM((G, 128), F32),
        ],
    )(s2a, s2b, dinv, bvec, batch_f, gw, gb, gms, S0, fcw8, fcb8)


def _make_comb_body(hdim, with_cnt, mode):
    def body(s0_ref, s1_ref, dinv_ref, b_ref, bf_ref, h_ref, S1_ref, S2_ref,
             *maybe_S0):
        i = pl.program_id(0)
        if mode == "cat":
            s = jnp.concatenate([s0_ref[...], s1_ref[...]], axis=1)
        else:
            s = s0_ref[...] + s1_ref[...]
        h = s * dinv_ref[...] + b_ref[...]
        h_ref[...] = h
        A = _onehot(bf_ref[...], BLK)
        dn = (((0,), (0,)), ((), ()))
        p1 = lax.dot_general(A, h, dn, preferred_element_type=F32, precision=lax.Precision.HIGHEST)
        p2 = lax.dot_general(A, h * h, dn, preferred_element_type=F32, precision=lax.Precision.HIGHEST)

        @pl.when(i == 0)
        def _():
            S1_ref[...] = jnp.zeros((G, hdim), F32)
            S2_ref[...] = jnp.zeros((G, hdim), F32)
            if with_cnt:
                maybe_S0[0][...] = jnp.zeros((G, 128), F32)

        S1_ref[...] += p1
        S2_ref[...] += p2
        if with_cnt:
            p0 = lax.dot_general(A, jnp.ones((BLK, 128), F32), dn,
                                 preferred_element_type=F32, precision=lax.Precision.HIGHEST)
            maybe_S0[0][...] += p0

    return body


def _run_comb(sa, sb, dinv, bvec, batch_f, hdim, with_cnt, mode):
    if mode == "cat":
        w = hdim // 2
        map_a = lambda i: (i, 0)
        map_b = lambda i: (NB + i, 0)
    else:
        w = hdim
        map_a = lambda i: (i, 0)
        map_b = lambda i: (i, 0)
    out_shape = [
        jax.ShapeDtypeStruct((N, hdim), F32),
        jax.ShapeDtypeStruct((G, hdim), F32),
        jax.ShapeDtypeStruct((G, hdim), F32),
    ]
    out_specs = [
        pl.BlockSpec((BLK, hdim), lambda i: (i, 0)),
        pl.BlockSpec((G, hdim), lambda i: (0, 0)),
        pl.BlockSpec((G, hdim), lambda i: (0, 0)),
    ]
    if with_cnt:
        out_shape.append(jax.ShapeDtypeStruct((G, 128), F32))
        out_specs.append(pl.BlockSpec((G, 128), lambda i: (0, 0)))
    return pl.pallas_call(
        _make_comb_body(hdim, with_cnt, mode),
        grid=(NB,),
        in_specs=[
            pl.BlockSpec((BLK, w), map_a),
            pl.BlockSpec((BLK, w), map_b),
            pl.BlockSpec((BLK, 1), lambda i: (i, 0)),
            pl.BlockSpec((1, hdim), lambda i: (0, 0)),
            pl.BlockSpec((BLK, 1), lambda i: (i, 0)),
        ],
        out_specs=out_specs,
        out_shape=out_shape,
    )(sa, sb, dinv, bvec, batch_f)


def _norm_relu(h, bf, S1, S2, S0, gw, gb, gms):
    cnt = jnp.maximum(S0[:, :1], 1.0)
    mean = S1 / cnt
    var = S2 / cnt + mean * mean * gms * (gms - 2.0)
    istd = lax.rsqrt(var + 1e-5)
    A = _onehot(bf, BLK)
    meanb = jnp.dot(A, gms * mean, preferred_element_type=F32, precision=lax.Precision.HIGHEST)
    istdb = jnp.dot(A, istd, preferred_element_type=F32, precision=lax.Precision.HIGHEST)
    hn = (h - meanb) * istdb * gw + gb
    return jnp.maximum(hn, 0.0)


def _c1_body(h_ref, bf_ref, S1_ref, S2_ref, S0_ref, gw_ref, gb_ref, gms_ref,
             dinv_ref, w_ref, y_ref):
    hr = _norm_relu(h_ref[...], bf_ref[...], S1_ref[...], S2_ref[...],
                    S0_ref[...], gw_ref[...], gb_ref[...], gms_ref[...])
    y_ref[...] = jnp.dot(hr * dinv_ref[...], w_ref[...],
                         preferred_element_type=F32, precision=lax.Precision.HIGHEST)


def _run_c1(h1, batch_f, S1, S2, S0, gw, gb, gms, dinv, W2):
    return pl.pallas_call(
        _c1_body,
        grid=(NB,),
        in_specs=[
            pl.BlockSpec((BLK, 256), lambda i: (i, 0)),
            pl.BlockSpec((BLK, 1), lambda i: (i, 0)),
            pl.BlockSpec((G, 256), lambda i: (0, 0)),
            pl.BlockSpec((G, 256), lambda i: (0, 0)),
            pl.BlockSpec((G, 128), lambda i: (0, 0)),
            pl.BlockSpec((1, 256), lambda i: (0, 0)),
            pl.BlockSpec((1, 256), lambda i: (0, 0)),
            pl.BlockSpec((1, 256), lambda i: (0, 0)),
            pl.BlockSpec((BLK, 1), lambda i: (i, 0)),
            pl.BlockSpec((256, 128), lambda i: (0, 0)),
        ],
        out_specs=pl.BlockSpec((BLK, 128), lambda i: (i, 0)),
        out_shape=jax.ShapeDtypeStruct((N, 128), F32),
    )(h1, batch_f, S1, S2, S0, gw, gb, gms, dinv, W2)


def _f_body(h_ref, bf_ref, S1_ref, S2_ref, S0_ref, gw_ref, gb_ref, gms_ref,
            w_ref, fb_ref, y_ref):
    hr = _norm_relu(h_ref[...], bf_ref[...], S1_ref[...], S2_ref[...],
                    S0_ref[...], gw_ref[...], gb_ref[...], gms_ref[...])
    y_ref[...] = jnp.dot(hr, w_ref[...], preferred_element_type=F32, precision=lax.Precision.HIGHEST) + fb_ref[...]


def _run_f(h2, batch_f, S1, S2, S0, gw, gb, gms, fcw8, fcb8):
    return pl.pallas_call(
        _f_body,
        grid=(NB,),
        in_specs=[
            pl.BlockSpec((BLK, 128), lambda i: (i, 0)),
            pl.BlockSpec((BLK, 1), lambda i: (i, 0)),
            pl.BlockSpec((G, 128), lambda i: (0, 0)),
            pl.BlockSpec((G, 128), lambda i: (0, 0)),
            pl.BlockSpec((G, 128), lambda i: (0, 0)),
            pl.BlockSpec((1, 128), lambda i: (0, 0)),
            pl.BlockSpec((1, 128), lambda i: (0, 0)),
            pl.BlockSpec((1, 128), lambda i: (0, 0)),
            pl.BlockSpec((128, 8), lambda i: (0, 0)),
            pl.BlockSpec((1, 8), lambda i: (0, 0)),
        ],
        out_specs=pl.BlockSpec((BLK, 8), lambda i: (i, 0)),
        out_shape=jax.ShapeDtypeStruct((N, 8), F32),
    )(h2, batch_f, S1, S2, S0, gw, gb, gms, fcw8, fcb8)


def kernel(x, index, batch, W1, b1, gn1_w, gn1_b, gn1_ms, W2, b2,
           gn2_w, gn2_b, gn2_ms, fc_W, fc_b):
    idx_flat = index.reshape(2 * E)
    batch_f = batch.astype(F32).reshape(N, 1)

    deg_a, deg_b = _deg_call(idx_flat)
    y1, dinv = _run_a1(x, W1, deg_a[:N].reshape(N, 1), deg_b[:N].reshape(N, 1))
    s1 = _scat128(y1, idx_flat)
    y2, S0 = _run_l1(s1, dinv, b1.reshape(1, 256), batch_f,
                     gn1_w.reshape(1, 256), gn1_b.reshape(1, 256),
                     gn1_ms.reshape(1, 256), W2)
    s2a, s2b = _scat_edge(y2, idx_flat)
    fcw8 = jnp.zeros((128, 8), F32).at[:, :2].set(fc_W)
    fcb8 = jnp.zeros((1, 8), F32).at[0, :2].set(fc_b)
    out8 = _run_l2(s2a, s2b, dinv, b2.reshape(1, 128), batch_f,
                   gn2_w.reshape(1, 128), gn2_b.reshape(1, 128),
                   gn2_ms.reshape(1, 128), S0, fcw8, fcb8)
    return out8[:, :2]


# speedup vs baseline: 1.4326x; 1.0700x over previous
import functools

import jax
import jax.numpy as jnp
from jax import lax
from jax.experimental import pallas as pl
from jax.experimental.pallas import tpu as pltpu
from jax.experimental.pallas import tpu_sc as plsc

N = 10000
E = 320000
G = 64
NB = 10
BLK = 1000
C = 80
NSUB = 16
F32 = jnp.float32

@functools.lru_cache(maxsize=None)
def _mesh():
    return plsc.VectorSubcoreMesh(core_axis_name="c", subcore_axis_name="s")


def _deg_body(idx_hbm, deg_a, deg_b, *scr):
    dst_v, isem, ssem = scr[0:4], scr[4:8], scr[8:12]
    ones_v, zbuf, acc = scr[12], scr[13], scr[14]
    cid = lax.axis_index("c")
    sid = lax.axis_index("s")
    for j in range(C // 16):
        ones_v[pl.ds(j * 16, 16)] = jnp.ones((16,), F32)
    for j in range(640 // 16):
        zbuf[pl.ds(j * 16, 16)] = jnp.zeros((16,), F32)

    @pl.when(sid < 15)
    def _():
        pltpu.sync_copy(zbuf, acc.at[pl.ds(sid * 640, 640)])

    @pl.when(sid == 15)
    def _():
        pltpu.sync_copy(zbuf.at[pl.ds(0, 400)], acc.at[pl.ds(9600, 400)])

    plsc.subcore_barrier()
    ebase = cid * (E // 2) + sid * (E // 2 // NSUB)
    nchunks = E // 2 // NSUB // C

    def start_idx(p, k):
        pltpu.async_copy(idx_hbm.at[pl.ds(E + ebase + k * C, C)], dst_v[p],
                         isem[p])

    def wait_idx(p, k):
        pltpu.make_async_copy(idx_hbm.at[pl.ds(E + ebase + k * C, C)],
                              dst_v[p], isem[p]).wait()

    def start_scatter(p):
        pltpu.async_copy(ones_v, acc.at[dst_v[p]], ssem[p], add=True)

    def wait_scatter(p):
        pltpu.make_async_copy(ones_v, acc.at[dst_v[p]], ssem[p]).wait()

    ngroups, rem = divmod(nchunks, NRING)
    start_idx(0, 0)

    def substep(k, p, t):
        p1 = (p + 1) % NRING
        if p < NRING - 1:
            @pl.when(t >= 1)
            def _():
                wait_scatter(p1)
        else:
            wait_scatter(p1)

        @pl.when(k + 1 < nchunks)
        def _():
            start_idx(p1, k + 1)

        wait_idx(p, k)
        start_scatter(p)

    def group(t, carry):
        for p in range(NRING):
            substep(NRING * t + p, p, t)
        return carry

    lax.fori_loop(0, ngroups, group, 0)
    for r in range(rem):
        k = ngroups * NRING + r
        p1 = (r + 1) % NRING
        wait_scatter(p1)
        if r + 1 < rem:
            start_idx(p1, k + 1)
        wait_idx(r, k)
        start_scatter(r)
    for d in (2, 1, 0):
        wait_scatter((nchunks - 1 - d) % NRING)
    plsc.subcore_barrier()

    def dump(out_ref, n):
        pltpu.sync_copy(acc.at[pl.ds(sid * 640, n)], zbuf.at[pl.ds(0, n)])
        pltpu.sync_copy(zbuf.at[pl.ds(0, n)], out_ref.at[pl.ds(sid * 640, n)])

    @pl.when(cid == 0)
    def _():
        @pl.when(sid < 15)
        def _():
            dump(deg_a, 640)

        @pl.when(sid == 15)
        def _():
            dump(deg_a, 400)

    @pl.when(cid == 1)
    def _():
        @pl.when(sid < 15)
        def _():
            dump(deg_b, 640)

        @pl.when(sid == 15)
        def _():
            dump(deg_b, 400)


def _deg_call(idx_flat):
    return pl.kernel(
        _deg_body,
        out_type=[jax.ShapeDtypeStruct((N,), F32),
                  jax.ShapeDtypeStruct((N,), F32)],
        mesh=_mesh(),
        scratch_types=(
            [pltpu.VMEM((C,), jnp.int32)] * NRING
            + [pltpu.SemaphoreType.DMA] * (2 * NRING)
            + [pltpu.VMEM((C,), F32),
               pltpu.VMEM((640,), F32),
               pltpu.VMEM_SHARED((N,), F32)]
        ),
    )(idx_flat)


NRING = 4


def _edge_pipeline(y_hbm, idx_hbm, acc, bufs, yoff, ebase, nchunks, do_off):
    (src_v, dst_v, rows_v, isem, gsem, ssem) = bufs
    if do_off:
        off = jnp.zeros((16,), jnp.int32) + yoff

    def start_idx(p, k):
        base = ebase + k * C
        pltpu.async_copy(idx_hbm.at[pl.ds(base, C)], src_v[p], isem[p])
        pltpu.async_copy(idx_hbm.at[pl.ds(E + base, C)], dst_v[p], isem[p])

    def launch_gather(p, k):
        base = ebase + k * C
        pltpu.make_async_copy(idx_hbm.at[pl.ds(base, C)], src_v[p],
                              isem[p]).wait()
        pltpu.make_async_copy(idx_hbm.at[pl.ds(E + base, C)], dst_v[p],
                              isem[p]).wait()
        if do_off:
            for j in range(C // 16):
                src_v[p][pl.ds(j * 16, 16)] = src_v[p][pl.ds(j * 16, 16)] + off
        pltpu.async_copy(y_hbm.at[src_v[p]], rows_v[p], gsem[p])

    def wait_gather(p):
        pltpu.make_async_copy(y_hbm.at[src_v[p]], rows_v[p], gsem[p]).wait()

    def start_scatter(p):
        pltpu.async_copy(rows_v[p], acc.at[dst_v[p]], ssem[p], add=True)

    def wait_scatter(p):
        pltpu.make_async_copy(rows_v[p], acc.at[dst_v[p]], ssem[p]).wait()

    ngroups, rem = divmod(nchunks, NRING)
    assert ngroups >= 1

    start_idx(0, 0)

    def substep(k, p, t):
        p1 = (p + 1) % NRING
        if p < NRING - 1:
            @pl.when(t >= 1)
            def _():
                wait_scatter(p1)
        else:
            wait_scatter(p1)

        @pl.when(k + 1 < nchunks)
        def _():
            start_idx(p1, k + 1)

        launch_gather(p, k)
        q = (p - 1) % NRING
        if p == 0:
            @pl.when(t >= 1)
            def _():
                wait_gather(q)
                start_scatter(q)
        else:
            wait_gather(q)
            start_scatter(q)

    def group(t, carry):
        for p in range(NRING):
            substep(NRING * t + p, p, t)
        return carry

    lax.fori_loop(0, ngroups, group, 0)
    for r in range(rem):
        k = ngroups * NRING + r
        p1 = (r + 1) % NRING
        wait_scatter(p1)
        if r + 1 < rem:
            start_idx(p1, k + 1)
        launch_gather(r, k)
        q = (r - 1) % NRING
        wait_gather(q)
        start_scatter(q)
    p_last = (nchunks - 1) % NRING
    wait_gather(p_last)
    start_scatter(p_last)
    for d in (3, 2, 1):
        wait_scatter((nchunks - d) % NRING)


def _stage_rows(nch, inner):
    def body(t, carry):
        inner(t)
        return carry

    lax.fori_loop(0, nch, body, 0)


def _make_scat(dh):
    def body(y_hbm, idx_hbm, out_hbm, *scr):
        src_v, dst_v, rows_v = scr[0:4], scr[4:8], scr[8:12]
        acc = scr[12]
        isem, gsem, ssem = scr[13:17], scr[17:21], scr[21:25]
        rows_a = rows_v[0]
        cid = lax.axis_index("c")
        sid = lax.axis_index("s")
        yoff = cid * N

        nch = jnp.where(sid == 15, 5, 8)

        def icopy(t):
            r0 = sid * 640 + t * C
            pltpu.sync_copy(y_hbm.at[pl.ds(yoff + r0, C)], rows_a)
            pltpu.sync_copy(rows_a, acc.at[pl.ds(r0, C)])

        _stage_rows(nch, icopy)
        plsc.subcore_barrier()

        bufs = (src_v, dst_v, rows_v, isem, gsem, ssem)
        _edge_pipeline(y_hbm, idx_hbm, acc, bufs, yoff,
                       sid * (E // NSUB), E // NSUB // C, True)
        plsc.subcore_barrier()

        def ocopy(t):
            r0 = sid * 640 + t * C
            pltpu.sync_copy(acc.at[pl.ds(r0, C)], rows_a)
            pltpu.sync_copy(rows_a, out_hbm.at[pl.ds(yoff + r0, C)])

        _stage_rows(nch, ocopy)

    def run(y, idx_flat):
        return pl.kernel(
            body,
            out_type=jax.ShapeDtypeStruct((2 * N, dh), F32),
            mesh=_mesh(),
            scratch_types=(
                [pltpu.VMEM((C,), jnp.int32)] * (2 * NRING)
                + [pltpu.VMEM((C, dh), F32)] * NRING
                + [pltpu.VMEM_SHARED((N, dh), F32)]
                + [pltpu.SemaphoreType.DMA] * (3 * NRING)
            ),
        )(y, idx_flat)

    return run


_scat128 = _make_scat(128)


def _scat_edge_body(y_hbm, idx_hbm, out_a, out_b, *scr):
    src_v, dst_v, rows_v = scr[0:4], scr[4:8], scr[8:12]
    acc = scr[12]
    isem, gsem, ssem = scr[13:17], scr[17:21], scr[21:25]
    rows_a = rows_v[0]
    cid = lax.axis_index("c")
    sid = lax.axis_index("s")
    nch = jnp.where(sid == 15, 5, 8)

    @pl.when(cid == 1)
    def _():
        def zrow(r, carry):
            for j in range(128 // 16):
                rows_a[r, pl.ds(j * 16, 16)] = jnp.zeros((16,), F32)
            return carry

        lax.fori_loop(0, C, zrow, 0)

    def icopy(t):
        r0 = sid * 640 + t * C

        @pl.when(cid == 0)
        def _():
            pltpu.sync_copy(y_hbm.at[pl.ds(r0, C)], rows_a)

        pltpu.sync_copy(rows_a, acc.at[pl.ds(r0, C)])

    _stage_rows(nch, icopy)
    plsc.subcore_barrier()

    bufs = (src_v, dst_v, rows_v, isem, gsem, ssem)
    _edge_pipeline(y_hbm, idx_hbm, acc, bufs, 0,
                   cid * (E // 2) + sid * (E // 2 // NSUB),
                   E // 2 // NSUB // C, False)
    plsc.subcore_barrier()

    def dump(out_ref):
        def ocopy(t):
            r0 = sid * 640 + t * C
            pltpu.sync_copy(acc.at[pl.ds(r0, C)], rows_a)
            pltpu.sync_copy(rows_a, out_ref.at[pl.ds(r0, C)])

        _stage_rows(nch, ocopy)

    @pl.when(cid == 0)
    def _():
        dump(out_a)

    @pl.when(cid == 1)
    def _():
        dump(out_b)


def _scat_edge(y, idx_flat):
    return pl.kernel(
        _scat_edge_body,
        out_type=[jax.ShapeDtypeStruct((N, 128), F32),
                  jax.ShapeDtypeStruct((N, 128), F32)],
        mesh=_mesh(),
        scratch_types=(
            [pltpu.VMEM((C,), jnp.int32)] * (2 * NRING)
            + [pltpu.VMEM((C, 128), F32)] * NRING
            + [pltpu.VMEM_SHARED((N, 128), F32)]
            + [pltpu.SemaphoreType.DMA] * (3 * NRING)
        ),
    )(y, idx_flat)


def _a1_body(x_ref, w_ref, da_ref, db_ref, y_ref, dinv_ref):
    dinv = lax.rsqrt(da_ref[...] + db_ref[...] + 1.0)
    dinv_ref[...] = dinv
    y_ref[...] = jnp.dot(x_ref[...] * dinv, w_ref[...],
                         preferred_element_type=F32, precision=lax.Precision.HIGHEST)


def _run_a1(x, W1, deg_a, deg_b):
    return pl.pallas_call(
        _a1_body,
        grid=(2, NB),
        in_specs=[
            pl.BlockSpec((BLK, 128), lambda h, i: (i, 0)),
            pl.BlockSpec((128, 128), lambda h, i: (0, h)),
            pl.BlockSpec((BLK, 1), lambda h, i: (i, 0)),
            pl.BlockSpec((BLK, 1), lambda h, i: (i, 0)),
        ],
        out_specs=[
            pl.BlockSpec((BLK, 128), lambda h, i: (h * NB + i, 0)),
            pl.BlockSpec((BLK, 1), lambda h, i: (i, 0)),
        ],
        out_shape=[
            jax.ShapeDtypeStruct((2 * N, 128), F32),
            jax.ShapeDtypeStruct((N, 1), F32),
        ],
    )(x, W1, deg_a, deg_b)


def _onehot(bcol, n_rows):
    iota = lax.broadcasted_iota(jnp.int32, (n_rows, G), 1).astype(F32)
    return (bcol == iota).astype(F32)


def _l1_fused_body(s0_ref, s1_ref, dinv_ref, b_ref, bf_ref, gw_ref, gb_ref,
                   gms_ref, w_ref, y_ref, S0_ref, hs, S1s, S2s):
    ph = pl.program_id(0)
    i = pl.program_id(1)
    dn = (((0,), (0,)), ((), ()))

    @pl.when(ph == 0)
    def _():
        h = (jnp.concatenate([s0_ref[...], s1_ref[...]], axis=1)
             * dinv_ref[...] + b_ref[...])
        hs[pl.ds(i * BLK, BLK), :] = h
        A = _onehot(bf_ref[...], BLK)
        p1 = lax.dot_general(A, h, dn, preferred_element_type=F32,
                             precision=lax.Precision.HIGHEST)
        p2 = lax.dot_general(A, h * h, dn, preferred_element_type=F32,
                             precision=lax.Precision.HIGHEST)
        p0 = lax.dot_general(A, jnp.ones((BLK, 128), F32), dn,
                             preferred_element_type=F32,
                             precision=lax.Precision.HIGHEST)

        @pl.when(i == 0)
        def _():
            S1s[...] = jnp.zeros((G, 256), F32)
            S2s[...] = jnp.zeros((G, 256), F32)
            S0_ref[...] = jnp.zeros((G, 128), F32)

        S1s[...] += p1
        S2s[...] += p2
        S0_ref[...] += p0

    @pl.when(ph == 1)
    def _():
        h = hs[pl.ds(i * BLK, BLK), :]
        hr = _norm_relu(h, bf_ref[...], S1s[...], S2s[...], S0_ref[...],
                        gw_ref[...], gb_ref[...], gms_ref[...])
        y_ref[...] = jnp.dot(hr * dinv_ref[...], w_ref[...],
                             preferred_element_type=F32,
                             precision=lax.Precision.HIGHEST)


def _run_l1(s1, dinv, bvec, batch_f, gw, gb, gms, W2):
    return pl.pallas_call(
        _l1_fused_body,
        grid=(2, NB),
        in_specs=[
            pl.BlockSpec((BLK, 128), lambda ph, i: (i, 0)),
            pl.BlockSpec((BLK, 128), lambda ph, i: (NB + i, 0)),
            pl.BlockSpec((BLK, 1), lambda ph, i: (i, 0)),
            pl.BlockSpec((1, 256), lambda ph, i: (0, 0)),
            pl.BlockSpec((BLK, 1), lambda ph, i: (i, 0)),
            pl.BlockSpec((1, 256), lambda ph, i: (0, 0)),
            pl.BlockSpec((1, 256), lambda ph, i: (0, 0)),
            pl.BlockSpec((1, 256), lambda ph, i: (0, 0)),
            pl.BlockSpec((256, 128), lambda ph, i: (0, 0)),
        ],
        out_specs=[
            pl.BlockSpec((BLK, 128), lambda ph, i: (i, 0)),
            pl.BlockSpec((G, 128), lambda ph, i: (0, 0)),
        ],
        out_shape=[
            jax.ShapeDtypeStruct((N, 128), F32),
            jax.ShapeDtypeStruct((G, 128), F32),
        ],
        scratch_shapes=[
            pltpu.VMEM((N, 256), F32),
            pltpu.VMEM((G, 256), F32),
            pltpu.VMEM((G, 256), F32),
        ],
    )(s1, s1, dinv, bvec, batch_f, gw, gb, gms, W2)


def _l2_fused_body(sa_ref, sb_ref, dinv_ref, b_ref, bf_ref, gw_ref, gb_ref,
                   gms_ref, S0_ref, w_ref, fb_ref, y_ref, hs, S1s, S2s):
    ph = pl.program_id(0)
    i = pl.program_id(1)
    dn = (((0,), (0,)), ((), ()))

    @pl.when(ph == 0)
    def _():
        h = (sa_ref[...] + sb_ref[...]) * dinv_ref[...] + b_ref[...]
        hs[pl.ds(i * BLK, BLK), :] = h
        A = _onehot(bf_ref[...], BLK)
        p1 = lax.dot_general(A, h, dn, preferred_element_type=F32,
                             precision=lax.Precision.HIGHEST)
        p2 = lax.dot_general(A, h * h, dn, preferred_element_type=F32,
                             precision=lax.Precision.HIGHEST)

        @pl.when(i == 0)
        def _():
            S1s[...] = jnp.zeros((G, 128), F32)
            S2s[...] = jnp.zeros((G, 128), F32)

        S1s[...] += p1
        S2s[...] += p2

    @pl.when(ph == 1)
    def _():
        h = hs[pl.ds(i * BLK, BLK), :]
        hr = _norm_relu(h, bf_ref[...], S1s[...], S2s[...], S0_ref[...],
                        gw_ref[...], gb_ref[...], gms_ref[...])
        y_ref[...] = jnp.dot(hr, w_ref[...], preferred_element_type=F32,
                             precision=lax.Precision.HIGHEST) + fb_ref[...]


def _run_l2(s2a, s2b, dinv, bvec, batch_f, gw, gb, gms, S0, fcw8, fcb8):
    return pl.pallas_call(
        _l2_fused_body,
        grid=(2, NB),
        in_specs=[
            pl.BlockSpec((BLK, 128), lambda ph, i: (i, 0)),
            pl.BlockSpec((BLK, 128), lambda ph, i: (i, 0)),
            pl.BlockSpec((BLK, 1), lambda ph, i: (i, 0)),
            pl.BlockSpec((1, 128), lambda ph, i: (0, 0)),
            pl.BlockSpec((BLK, 1), lambda ph, i: (i, 0)),
            pl.BlockSpec((1, 128), lambda ph, i: (0, 0)),
            pl.BlockSpec((1, 128), lambda ph, i: (0, 0)),
            pl.BlockSpec((1, 128), lambda ph, i: (0, 0)),
            pl.BlockSpec((G, 128), lambda ph, i: (0, 0)),
            pl.BlockSpec((128, 8), lambda ph, i: (0, 0)),
            pl.BlockSpec((1, 8), lambda ph, i: (0, 0)),
        ],
        out_specs=pl.BlockSpec((BLK, 8), lambda ph, i: (i, 0)),
        out_shape=jax.ShapeDtypeStruct((N, 8), F32),
        scratch_shapes=[
            pltpu.VMEM((N, 128), F32),
            pltpu.VMEM((G, 128), F32),
            pltpu.VMEM((G, 128), F32),
        ],
    )(s2a, s2b, dinv, bvec, batch_f, gw, gb, gms, S0, fcw8, fcb8)


def _make_comb_body(hdim, with_cnt, mode):
    def body(s0_ref, s1_ref, dinv_ref, b_ref, bf_ref, h_ref, S1_ref, S2_ref,
             *maybe_S0):
        i = pl.program_id(0)
        if mode == "cat":
            s = jnp.concatenate([s0_ref[...], s1_ref[...]], axis=1)
        else:
            s = s0_ref[...] + s1_ref[...]
        h = s * dinv_ref[...] + b_ref[...]
        h_ref[...] = h
        A = _onehot(bf_ref[...], BLK)
        dn = (((0,), (0,)), ((), ()))
        p1 = lax.dot_general(A, h, dn, preferred_element_type=F32, precision=lax.Precision.HIGHEST)
        p2 = lax.dot_general(A, h * h, dn, preferred_element_type=F32, precision=lax.Precision.HIGHEST)

        @pl.when(i == 0)
        def _():
            S1_ref[...] = jnp.zeros((G, hdim), F32)
            S2_ref[...] = jnp.zeros((G, hdim), F32)
            if with_cnt:
                maybe_S0[0][...] = jnp.zeros((G, 128), F32)

        S1_ref[...] += p1
        S2_ref[...] += p2
        if with_cnt:
            p0 = lax.dot_general(A, jnp.ones((BLK, 128), F32), dn,
                                 preferred_element_type=F32, precision=lax.Precision.HIGHEST)
            maybe_S0[0][...] += p0

    return body


def _run_comb(sa, sb, dinv, bvec, batch_f, hdim, with_cnt, mode):
    if mode == "cat":
        w = hdim // 2
        map_a = lambda i: (i, 0)
        map_b = lambda i: (NB + i, 0)
    else:
        w = hdim
        map_a = lambda i: (i, 0)
        map_b = lambda i: (i, 0)
    out_shape = [
        jax.ShapeDtypeStruct((N, hdim), F32),
        jax.ShapeDtypeStruct((G, hdim), F32),
        jax.ShapeDtypeStruct((G, hdim), F32),
    ]
    out_specs = [
        pl.BlockSpec((BLK, hdim), lambda i: (i, 0)),
        pl.BlockSpec((G, hdim), lambda i: (0, 0)),
        pl.BlockSpec((G, hdim), lambda i: (0, 0)),
    ]
    if with_cnt:
        out_shape.append(jax.ShapeDtypeStruct((G, 128), F32))
        out_specs.append(pl.BlockSpec((G, 128), lambda i: (0, 0)))
    return pl.pallas_call(
        _make_comb_body(hdim, with_cnt, mode),
        grid=(NB,),
        in_specs=[
            pl.BlockSpec((BLK, w), map_a),
            pl.BlockSpec((BLK, w), map_b),
            pl.BlockSpec((BLK, 1), lambda i: (i, 0)),
            pl.BlockSpec((1, hdim), lambda i: (0, 0)),
            pl.BlockSpec((BLK, 1), lambda i: (i, 0)),
        ],
        out_specs=out_specs,
        out_shape=out_shape,
    )(sa, sb, dinv, bvec, batch_f)


def _norm_relu(h, bf, S1, S2, S0, gw, gb, gms):
    cnt = jnp.maximum(S0[:, :1], 1.0)
    mean = S1 / cnt
    var = S2 / cnt + mean * mean * gms * (gms - 2.0)
    istd = lax.rsqrt(var + 1e-5)
    A = _onehot(bf, BLK)
    meanb = jnp.dot(A, gms * mean, preferred_element_type=F32, precision=lax.Precision.HIGHEST)
    istdb = jnp.dot(A, istd, preferred_element_type=F32, precision=lax.Precision.HIGHEST)
    hn = (h - meanb) * istdb * gw + gb
    return jnp.maximum(hn, 0.0)


def _c1_body(h_ref, bf_ref, S1_ref, S2_ref, S0_ref, gw_ref, gb_ref, gms_ref,
             dinv_ref, w_ref, y_ref):
    hr = _norm_relu(h_ref[...], bf_ref[...], S1_ref[...], S2_ref[...],
                    S0_ref[...], gw_ref[...], gb_ref[...], gms_ref[...])
    y_ref[...] = jnp.dot(hr * dinv_ref[...], w_ref[...],
                         preferred_element_type=F32, precision=lax.Precision.HIGHEST)


def _run_c1(h1, batch_f, S1, S2, S0, gw, gb, gms, dinv, W2):
    return pl.pallas_call(
        _c1_body,
        grid=(NB,),
        in_specs=[
            pl.BlockSpec((BLK, 256), lambda i: (i, 0)),
            pl.BlockSpec((BLK, 1), lambda i: (i, 0)),
            pl.BlockSpec((G, 256), lambda i: (0, 0)),
            pl.BlockSpec((G, 256), lambda i: (0, 0)),
            pl.BlockSpec((G, 128), lambda i: (0, 0)),
            pl.BlockSpec((1, 256), lambda i: (0, 0)),
            pl.BlockSpec((1, 256), lambda i: (0, 0)),
            pl.BlockSpec((1, 256), lambda i: (0, 0)),
            pl.BlockSpec((BLK, 1), lambda i: (i, 0)),
            pl.BlockSpec((256, 128), lambda i: (0, 0)),
        ],
        out_specs=pl.BlockSpec((BLK, 128), lambda i: (i, 0)),
        out_shape=jax.ShapeDtypeStruct((N, 128), F32),
    )(h1, batch_f, S1, S2, S0, gw, gb, gms, dinv, W2)


def _f_body(h_ref, bf_ref, S1_ref, S2_ref, S0_ref, gw_ref, gb_ref, gms_ref,
            w_ref, fb_ref, y_ref):
    hr = _norm_relu(h_ref[...], bf_ref[...], S1_ref[...], S2_ref[...],
                    S0_ref[...], gw_ref[...], gb_ref[...], gms_ref[...])
    y_ref[...] = jnp.dot(hr, w_ref[...], preferred_element_type=F32, precision=lax.Precision.HIGHEST) + fb_ref[...]


def _run_f(h2, batch_f, S1, S2, S0, gw, gb, gms, fcw8, fcb8):
    return pl.pallas_call(
        _f_body,
        grid=(NB,),
        in_specs=[
            pl.BlockSpec((BLK, 128), lambda i: (i, 0)),
            pl.BlockSpec((BLK, 1), lambda i: (i, 0)),
            pl.BlockSpec((G, 128), lambda i: (0, 0)),
            pl.BlockSpec((G, 128), lambda i: (0, 0)),
            pl.BlockSpec((G, 128), lambda i: (0, 0)),
            pl.BlockSpec((1, 128), lambda i: (0, 0)),
            pl.BlockSpec((1, 128), lambda i: (0, 0)),
            pl.BlockSpec((1, 128), lambda i: (0, 0)),
            pl.BlockSpec((128, 8), lambda i: (0, 0)),
            pl.BlockSpec((1, 8), lambda i: (0, 0)),
        ],
        out_specs=pl.BlockSpec((BLK, 8), lambda i: (i, 0)),
        out_shape=jax.ShapeDtypeStruct((N, 8), F32),
    )(h2, batch_f, S1, S2, S0, gw, gb, gms, fcw8, fcb8)


def kernel(x, index, batch, W1, b1, gn1_w, gn1_b, gn1_ms, W2, b2,
           gn2_w, gn2_b, gn2_ms, fc_W, fc_b):
    idx_flat = index.reshape(2 * E)
    batch_f = batch.astype(F32).reshape(N, 1)

    deg_a, deg_b = _deg_call(idx_flat)
    y1, dinv = _run_a1(x, W1, deg_a[:N].reshape(N, 1), deg_b[:N].reshape(N, 1))
    s1 = _scat128(y1, idx_flat)
    y2, S0 = _run_l1(s1, dinv, b1.reshape(1, 256), batch_f,
                     gn1_w.reshape(1, 256), gn1_b.reshape(1, 256),
                     gn1_ms.reshape(1, 256), W2)
    s2a, s2b = _scat_edge(y2, idx_flat)
    fcw8 = jnp.zeros((128, 8), F32).at[:, :2].set(fc_W)
    fcb8 = jnp.zeros((1, 8), F32).at[0, :2].set(fc_b)
    out8 = _run_l2(s2a, s2b, dinv, b2.reshape(1, 128), batch_f,
                   gn2_w.reshape(1, 128), gn2_b.reshape(1, 128),
                   gn2_ms.reshape(1, 128), S0, fcw8, fcb8)
    return out8[:, :2]
